# Initial kernel scaffold; baseline (speedup 1.0000x reference)
import jax
import jax.numpy as jnp
from jax.experimental import pallas as pl


def kernel(x, edge_index, edge_attr, pos_edge_index, pos_edge_attr, neg_edge_index, neg_edge_attr, params):
    raise NotImplementedError("write your pallas kernel here")


# speedup vs baseline: 2.7674x; 2.7674x over previous
import functools

import jax
import jax.numpy as jnp
from jax import lax
from jax.experimental import pallas as pl
from jax.experimental.pallas import tpu as pltpu
from jax.experimental.pallas import tpu_sc as plsc

N = 10000
E = 320000
EP = 65536
H = 128
DE = 16

NC, NS = 2, 16
NW = NC * NS
EW = E // NW
KE = 80
NCH = EW // KE
GROUPS = H // 16
ROWCH = 80
NZCH = N // ROWCH
ZITER = (NZCH + NS - 1) // NS

K2 = 128
EW2 = 2 * EP // NW
NCH2 = EW2 // K2


@functools.lru_cache(maxsize=None)
def _sc_mesh():
    return plsc.VectorSubcoreMesh(core_axis_name="c", subcore_axis_name="s",
                                  num_cores=NC, num_subcores=NS)


@functools.lru_cache(maxsize=None)
def _sc_message_kernel():
    @functools.partial(
        pl.kernel,
        out_type=jax.ShapeDtypeStruct((NC * N, H), jnp.float32),
        mesh=_sc_mesh(),
        scratch_types=[
            pltpu.VMEM((KE,), jnp.int32),
            pltpu.VMEM((KE,), jnp.int32),
            pltpu.VMEM((KE, H), jnp.float32),
            pltpu.VMEM((KE, H), jnp.float32),
            pltpu.VMEM_SHARED((N, H), jnp.float32),
            pltpu.SemaphoreType.DMA,
        ],
    )
    def msg(h_hbm, ea_hbm, src_hbm, dst_hbm, out_hbm,
            src_v, dst_v, rows_v, ea_v, acc_sh, sem):
        cid = lax.axis_index("c")
        sid = lax.axis_index("s")
        wid = sid * NC + cid

        def zrow(r, _):
            for g in range(GROUPS):
                rows_v[r, pl.ds(g * 16, 16)] = jnp.zeros((16,), jnp.float32)
            return 0
        lax.fori_loop(0, KE, zrow, 0)

        def zchunk(t, _):
            c = sid + t * NS

            @pl.when(c < NZCH)
            def _():
                pltpu.sync_copy(rows_v, acc_sh.at[pl.ds(c * ROWCH, ROWCH)])
            return 0
        lax.fori_loop(0, ZITER, zchunk, 0)
        plsc.subcore_barrier()

        def echunk(ch, _):
            base = wid * EW + ch * KE
            pltpu.sync_copy(src_hbm.at[pl.ds(base, KE)], src_v)
            pltpu.sync_copy(dst_hbm.at[pl.ds(base, KE)], dst_v)
            pltpu.async_copy(h_hbm.at[src_v], rows_v, sem).wait()
            pltpu.sync_copy(ea_hbm.at[pl.ds(base, KE)], ea_v)

            def rbody(r, _):
                for g in range(GROUPS):
                    sl = pl.ds(g * 16, 16)
                    rows_v[r, sl] = jnp.maximum(rows_v[r, sl] + ea_v[r, sl],
                                                0.0)
                return 0
            lax.fori_loop(0, KE, rbody, 0)
            pltpu.sync_copy(rows_v, acc_sh.at[dst_v], add=True)
            return 0
        lax.fori_loop(0, NCH, echunk, 0)
        plsc.subcore_barrier()

        def wchunk(t, _):
            c = sid + t * NS

            @pl.when(c < NZCH)
            def _():
                off = cid * N + c * ROWCH
                pltpu.sync_copy(acc_sh.at[pl.ds(c * ROWCH, ROWCH)],
                                out_hbm.at[pl.ds(off, ROWCH)])
            return 0
        lax.fori_loop(0, ZITER, wchunk, 0)
    return msg


@functools.lru_cache(maxsize=None)
def _sc_gather2_kernel():
    @functools.partial(
        pl.kernel,
        out_type=jax.ShapeDtypeStruct((2 * EP, H), jnp.float32),
        mesh=_sc_mesh(),
        scratch_types=[
            pltpu.VMEM((K2,), jnp.int32),
            pltpu.VMEM((K2,), jnp.int32),
            pltpu.VMEM((K2, H), jnp.float32),
            pltpu.VMEM((K2, H), jnp.float32),
            pltpu.SemaphoreType.DMA,
            pltpu.SemaphoreType.DMA,
        ],
    )
    def gat(a_hbm, b_hbm, s_hbm, d_hbm, out_hbm,
            si_v, di_v, ra_v, rb_v, sem1, sem2):
        cid = lax.axis_index("c")
        sid = lax.axis_index("s")
        wid = sid * NC + cid

        def chunk(ch, _):
            base = wid * EW2 + ch * K2
            pltpu.sync_copy(s_hbm.at[pl.ds(base, K2)], si_v)
            pltpu.sync_copy(d_hbm.at[pl.ds(base, K2)], di_v)
            cpa = pltpu.async_copy(a_hbm.at[si_v], ra_v, sem1)
            cpb = pltpu.async_copy(b_hbm.at[di_v], rb_v, sem2)
            cpa.wait()
            cpb.wait()

            def rbody(r, _):
                for g in range(GROUPS):
                    sl = pl.ds(g * 16, 16)
                    ra_v[r, sl] = ra_v[r, sl] + rb_v[r, sl]
                return 0
            lax.fori_loop(0, K2, rbody, 0)
            pltpu.sync_copy(ra_v, out_hbm.at[pl.ds(base, K2)])
            return 0
        lax.fori_loop(0, NCH2, chunk, 0)
    return gat


def _mm_body(x_ref, w_ref, b_ref, o_ref):
    o_ref[...] = (jnp.dot(x_ref[...], w_ref[...],
                          preferred_element_type=jnp.float32) + b_ref[...])


def _mm(x, w, b, bm):
    m, k = x.shape
    n = w.shape[1]
    return pl.pallas_call(
        _mm_body,
        grid=(m // bm,),
        in_specs=[pl.BlockSpec((bm, k), lambda i: (i, 0)),
                  pl.BlockSpec((k, n), lambda i: (0, 0)),
                  pl.BlockSpec((1, n), lambda i: (0, 0))],
        out_specs=pl.BlockSpec((bm, n), lambda i: (i, 0)),
        out_shape=jax.ShapeDtypeStruct((m, n), jnp.float32),
    )(x, w, b)


def _layer_body(h_ref, p_ref, w1_ref, b1_ref, w2_ref, b2_ref, g_ref, bb_ref,
                o_ref):
    h = h_ref[...]
    z0 = h + p_ref[0:N, :] + p_ref[N:2 * N, :]
    z1 = jnp.maximum(jnp.dot(z0, w1_ref[...],
                             preferred_element_type=jnp.float32) + b1_ref[...],
                     0.0)
    z2 = jnp.dot(z1, w2_ref[...],
                 preferred_element_type=jnp.float32) + b2_ref[...]
    mu = jnp.mean(z2, axis=0, keepdims=True)
    var = jnp.mean((z2 - mu) ** 2, axis=0, keepdims=True)
    zn = (z2 - mu) / jnp.sqrt(var + 1e-5) * g_ref[...] + bb_ref[...]
    o_ref[...] = (h + jnp.maximum(zn, 0.0)) * 0.5


def _layer(h, parts, w1, b1, w2, b2, g, bb):
    return pl.pallas_call(
        _layer_body,
        out_shape=jax.ShapeDtypeStruct((N, H), jnp.float32),
    )(h, parts, w1, b1[None, :], w2, b2[None, :], g[None, :], bb[None, :])


def _ab_body(h_ref, wa_ref, wb_ref, a_ref, b_ref):
    hr = jnp.maximum(h_ref[...], 0.0)
    a_ref[...] = jnp.dot(hr, wa_ref[...], preferred_element_type=jnp.float32)
    b_ref[...] = jnp.dot(hr, wb_ref[...], preferred_element_type=jnp.float32)


def _ab(h, wa, wb):
    return pl.pallas_call(
        _ab_body,
        out_shape=(jax.ShapeDtypeStruct((N, H), jnp.float32),
                   jax.ShapeDtypeStruct((N, H), jnp.float32)),
    )(h, wa, wb)


def _fold_body(we_ref, be_ref, w1c_ref, rb1_ref, wcp_ref, bcp_ref):
    wcp_ref[...] = jnp.dot(we_ref[...], w1c_ref[...],
                           preferred_element_type=jnp.float32)
    bcp_ref[...] = jnp.dot(be_ref[...], w1c_ref[...],
                           preferred_element_type=jnp.float32) + rb1_ref[...]


def _fold(we, be, w1c, rb1):
    return pl.pallas_call(
        _fold_body,
        out_shape=(jax.ShapeDtypeStruct((DE, H), jnp.float32),
                   jax.ShapeDtypeStruct((1, H), jnp.float32)),
    )(we, be, w1c, rb1)


def _head_body(g_ref, c_ref, w2_ref, b2_ref, w3_ref, b3_ref, o_ref):
    z = jnp.maximum(g_ref[...] + c_ref[...], 0.0)
    z = jnp.maximum(jnp.dot(z, w2_ref[...],
                            preferred_element_type=jnp.float32) + b2_ref[...],
                    0.0)
    s = jnp.dot(z, w3_ref[...], preferred_element_type=jnp.float32) + b3_ref[...]
    o_ref[...] = 1.0 / (1.0 + jnp.exp(-s))


def _head(g, c, w2p, b2p, w3p, b3p, bm):
    m = g.shape[0]
    return pl.pallas_call(
        _head_body,
        grid=(m // bm,),
        in_specs=[pl.BlockSpec((bm, H), lambda i: (i, 0)),
                  pl.BlockSpec((bm, H), lambda i: (i, 0)),
                  pl.BlockSpec((H, 32), lambda i: (0, 0)),
                  pl.BlockSpec((1, 32), lambda i: (0, 0)),
                  pl.BlockSpec((32, 8), lambda i: (0, 0)),
                  pl.BlockSpec((1, 8), lambda i: (0, 0))],
        out_specs=pl.BlockSpec((bm, 8), lambda i: (i, 0)),
        out_shape=jax.ShapeDtypeStruct((m, 8), jnp.float32),
    )(g, c, w2p, b2p, w3p, b3p)


@jax.jit
def kernel(x, edge_index, edge_attr, pos_edge_index, pos_edge_attr,
           neg_edge_index, neg_edge_attr, params):
    p = params
    h = _mm(x, p['node_W'], p['node_b'][None, :], 2000)
    ea = _mm(edge_attr, p['edge_W'], p['edge_b'][None, :], 8000)
    src = edge_index[0]
    dst = edge_index[1]

    w1a = p['r_W1'][0:H, :]
    w1b = p['r_W1'][H:2 * H, :]
    w1c = p['r_W1'][2 * H:3 * H, :]
    wcp, bcp = _fold(p['edge_W'], p['edge_b'][None, :], w1c, p['r_b1'][None, :])
    attr2 = jnp.concatenate([pos_edge_attr, neg_edge_attr], axis=0)
    cc = _mm(attr2, wcp, bcp, 8192)

    msg = _sc_message_kernel()
    for i in range(2):
        parts = msg(h, ea, src, dst)
        h = _layer(h, parts, p[f'conv{i}_W1'], p[f'conv{i}_b1'],
                   p[f'conv{i}_W2'], p[f'conv{i}_b2'],
                   p[f'bn{i}_g'], p[f'bn{i}_b'])

    a, b = _ab(h, w1a, w1b)
    ei2 = jnp.concatenate([pos_edge_index, neg_edge_index], axis=1)
    g = _sc_gather2_kernel()(a, b, ei2[0], ei2[1])

    w2p = jnp.pad(p['r_W2'], ((0, 0), (0, 7)))
    b2p = jnp.pad(p['r_b2'], (0, 7))[None, :]
    w3p = jnp.pad(p['r_W3'], ((0, 7), (0, 7)))
    b3p = jnp.pad(p['r_b3'], (0, 7))[None, :]
    out = _head(g, cc, w2p, b2p, w3p, b3p, 8192)
    return out[:EP, 0:1], out[EP:, 0:1], h


# speedup vs baseline: 4.1420x; 1.4967x over previous
import functools

import jax
import jax.numpy as jnp
from jax import lax
from jax.experimental import pallas as pl
from jax.experimental.pallas import tpu as pltpu
from jax.experimental.pallas import tpu_sc as plsc

N = 10000
E = 320000
EP = 65536
H = 128
DE = 16

NC, NS = 2, 16
NW = NC * NS
EW = E // NW
KE = 40
NCH = EW // KE
GROUPS = H // 16
ROWCH = 80
NZCH = N // ROWCH
ZITER = (NZCH + NS - 1) // NS

K2 = 128
EW2 = 2 * EP // NW
NCH2 = EW2 // K2


@functools.lru_cache(maxsize=None)
def _sc_mesh():
    return plsc.VectorSubcoreMesh(core_axis_name="c", subcore_axis_name="s",
                                  num_cores=NC, num_subcores=NS)


@functools.lru_cache(maxsize=None)
def _sc_message_kernel():
    @functools.partial(
        pl.kernel,
        out_type=jax.ShapeDtypeStruct((NC * N, H), jnp.float32),
        mesh=_sc_mesh(),
        scratch_types=[
            pltpu.VMEM((4, KE), jnp.int32),
            pltpu.VMEM((4, KE), jnp.int32),
            pltpu.VMEM((2, KE, H), jnp.float32),
            pltpu.VMEM((2, KE, H), jnp.float32),
            pltpu.VMEM_SHARED((N, H), jnp.float32),
            pltpu.SemaphoreType.DMA,
            pltpu.SemaphoreType.DMA,
            pltpu.SemaphoreType.DMA,
            pltpu.SemaphoreType.DMA,
            pltpu.SemaphoreType.DMA,
            pltpu.SemaphoreType.DMA,
            pltpu.SemaphoreType.DMA,
            pltpu.SemaphoreType.DMA,
        ],
    )
    def msg(h_hbm, ea_hbm, src_hbm, dst_hbm, out_hbm,
            src_v, dst_v, rows_v, ea_v, acc_sh,
            si0, si1, si2, si3, sg0, sg1, se0, se1):
        cid = lax.axis_index("c")
        sid = lax.axis_index("s")
        wid = sid * NC + cid
        sis = (si0, si1, si2, si3)
        sgs = (sg0, sg1)
        ses = (se0, se1)

        def zrow(r, _):
            for g in range(GROUPS):
                rows_v[0, r, pl.ds(g * 16, 16)] = jnp.zeros((16,), jnp.float32)
            return 0
        lax.fori_loop(0, KE, zrow, 0)

        def zchunk(t, _):
            c = sid + t * NS

            @pl.when(c < NZCH)
            def _():
                for rep in range(ROWCH // KE):
                    pltpu.sync_copy(
                        rows_v.at[0],
                        acc_sh.at[pl.ds(c * ROWCH + rep * KE, KE)])
            return 0
        lax.fori_loop(0, ZITER, zchunk, 0)
        plsc.subcore_barrier()

        def issue_idx(c, s):
            base = wid * EW + c * KE
            pltpu.async_copy(src_hbm.at[pl.ds(base, KE)], src_v.at[s], sis[s])
            pltpu.async_copy(dst_hbm.at[pl.ds(base, KE)], dst_v.at[s], sis[s])

        def wait_idx(s):
            pltpu.make_async_copy(src_hbm.at[pl.ds(0, KE)], src_v.at[s],
                                  sis[s]).wait()
            pltpu.make_async_copy(dst_hbm.at[pl.ds(0, KE)], dst_v.at[s],
                                  sis[s]).wait()

        def issue_data(c, i, d):
            base = wid * EW + c * KE
            pltpu.async_copy(h_hbm.at[src_v.at[i]], rows_v.at[d], sgs[d])
            pltpu.async_copy(ea_hbm.at[pl.ds(base, KE)], ea_v.at[d], ses[d])

        def wait_data(i, d):
            pltpu.make_async_copy(h_hbm.at[src_v.at[i]], rows_v.at[d],
                                  sgs[d]).wait()
            pltpu.make_async_copy(ea_hbm.at[pl.ds(0, KE)], ea_v.at[d],
                                  ses[d]).wait()

        def process(i, d):
            def rbody(r, _):
                for g in range(GROUPS):
                    sl = pl.ds(g * 16, 16)
                    rows_v[d, r, sl] = jnp.maximum(
                        rows_v[d, r, sl] + ea_v[d, r, sl], 0.0)
                return 0
            lax.fori_loop(0, KE, rbody, 0)
            pltpu.sync_copy(rows_v.at[d], acc_sh.at[dst_v.at[i]], add=True)

        NQ = NCH // 4
        PIPE = NQ * 4

        for s in range(4):
            pltpu.sync_copy(src_hbm.at[pl.ds(wid * EW + s * KE, KE)],
                            src_v.at[s])
            pltpu.sync_copy(dst_hbm.at[pl.ds(wid * EW + s * KE, KE)],
                            dst_v.at[s])
        issue_data(0, 0, 0)

        def quad(u, _):
            c0 = u * 4
            more = u < NQ - 1
            issue_data(c0 + 1, 1, 1)
            wait_data(0, 0)
            process(0, 0)

            @pl.when(more)
            def _():
                issue_idx(c0 + 4, 0)
            issue_data(c0 + 2, 2, 0)
            wait_data(1, 1)
            process(1, 1)

            @pl.when(more)
            def _():
                issue_idx(c0 + 5, 1)
            issue_data(c0 + 3, 3, 1)
            wait_data(2, 0)
            process(2, 0)

            @pl.when(more)
            def _():
                issue_idx(c0 + 6, 2)
                wait_idx(0)
                issue_data(c0 + 4, 0, 0)
            wait_data(3, 1)
            process(3, 1)

            @pl.when(more)
            def _():
                issue_idx(c0 + 7, 3)
                wait_idx(1)
                wait_idx(2)
                wait_idx(3)
            return 0
        lax.fori_loop(0, NQ, quad, 0)

        def tail(c, _):
            base = wid * EW + c * KE
            pltpu.sync_copy(src_hbm.at[pl.ds(base, KE)], src_v.at[0])
            pltpu.sync_copy(dst_hbm.at[pl.ds(base, KE)], dst_v.at[0])
            pltpu.async_copy(h_hbm.at[src_v.at[0]], rows_v.at[0], sg0).wait()
            pltpu.sync_copy(ea_hbm.at[pl.ds(base, KE)], ea_v.at[0])
            process(0, 0)
            return 0
        if PIPE < NCH:
            lax.fori_loop(PIPE, NCH, tail, 0)
        plsc.subcore_barrier()

        def wchunk(t, _):
            c = sid + t * NS

            @pl.when(c < NZCH)
            def _():
                off = cid * N + c * ROWCH
                for rep in range(ROWCH // KE):
                    pltpu.sync_copy(
                        acc_sh.at[pl.ds(c * ROWCH + rep * KE, KE)],
                        out_hbm.at[pl.ds(off + rep * KE, KE)])
            return 0
        lax.fori_loop(0, ZITER, wchunk, 0)
    return msg


@functools.lru_cache(maxsize=None)
def _sc_gather2_kernel():
    @functools.partial(
        pl.kernel,
        out_type=jax.ShapeDtypeStruct((2 * EP, H), jnp.float32),
        mesh=_sc_mesh(),
        scratch_types=[
            pltpu.VMEM((K2,), jnp.int32),
            pltpu.VMEM((K2,), jnp.int32),
            pltpu.VMEM((K2, H), jnp.float32),
            pltpu.VMEM((K2, H), jnp.float32),
            pltpu.SemaphoreType.DMA,
            pltpu.SemaphoreType.DMA,
        ],
    )
    def gat(a_hbm, b_hbm, s_hbm, d_hbm, out_hbm,
            si_v, di_v, ra_v, rb_v, sem1, sem2):
        cid = lax.axis_index("c")
        sid = lax.axis_index("s")
        wid = sid * NC + cid

        def chunk(ch, _):
            base = wid * EW2 + ch * K2
            pltpu.sync_copy(s_hbm.at[pl.ds(base, K2)], si_v)
            pltpu.sync_copy(d_hbm.at[pl.ds(base, K2)], di_v)
            cpa = pltpu.async_copy(a_hbm.at[si_v], ra_v, sem1)
            cpb = pltpu.async_copy(b_hbm.at[di_v], rb_v, sem2)
            cpa.wait()
            cpb.wait()

            def rbody(r, _):
                for g in range(GROUPS):
                    sl = pl.ds(g * 16, 16)
                    ra_v[r, sl] = ra_v[r, sl] + rb_v[r, sl]
                return 0
            lax.fori_loop(0, K2, rbody, 0)
            pltpu.sync_copy(ra_v, out_hbm.at[pl.ds(base, K2)])
            return 0
        lax.fori_loop(0, NCH2, chunk, 0)
    return gat


def _mm_body(x_ref, w_ref, b_ref, o_ref):
    o_ref[...] = (jnp.dot(x_ref[...], w_ref[...],
                          preferred_element_type=jnp.float32) + b_ref[...])


def _mm(x, w, b, bm):
    m, k = x.shape
    n = w.shape[1]
    return pl.pallas_call(
        _mm_body,
        grid=(m // bm,),
        in_specs=[pl.BlockSpec((bm, k), lambda i: (i, 0)),
                  pl.BlockSpec((k, n), lambda i: (0, 0)),
                  pl.BlockSpec((1, n), lambda i: (0, 0))],
        out_specs=pl.BlockSpec((bm, n), lambda i: (i, 0)),
        out_shape=jax.ShapeDtypeStruct((m, n), jnp.float32),
    )(x, w, b)


def _layer_body(h_ref, p_ref, w1_ref, b1_ref, w2_ref, b2_ref, g_ref, bb_ref,
                o_ref):
    h = h_ref[...]
    z0 = h + p_ref[0:N, :] + p_ref[N:2 * N, :]
    z1 = jnp.maximum(jnp.dot(z0, w1_ref[...],
                             preferred_element_type=jnp.float32) + b1_ref[...],
                     0.0)
    z2 = jnp.dot(z1, w2_ref[...],
                 preferred_element_type=jnp.float32) + b2_ref[...]
    mu = jnp.mean(z2, axis=0, keepdims=True)
    var = jnp.mean((z2 - mu) ** 2, axis=0, keepdims=True)
    zn = (z2 - mu) / jnp.sqrt(var + 1e-5) * g_ref[...] + bb_ref[...]
    o_ref[...] = (h + jnp.maximum(zn, 0.0)) * 0.5


def _layer(h, parts, w1, b1, w2, b2, g, bb):
    return pl.pallas_call(
        _layer_body,
        out_shape=jax.ShapeDtypeStruct((N, H), jnp.float32),
    )(h, parts, w1, b1[None, :], w2, b2[None, :], g[None, :], bb[None, :])


def _ab_body(h_ref, wa_ref, wb_ref, a_ref, b_ref):
    hr = jnp.maximum(h_ref[...], 0.0)
    a_ref[...] = jnp.dot(hr, wa_ref[...], preferred_element_type=jnp.float32)
    b_ref[...] = jnp.dot(hr, wb_ref[...], preferred_element_type=jnp.float32)


def _ab(h, wa, wb):
    return pl.pallas_call(
        _ab_body,
        out_shape=(jax.ShapeDtypeStruct((N, H), jnp.float32),
                   jax.ShapeDtypeStruct((N, H), jnp.float32)),
    )(h, wa, wb)


def _fold_body(we_ref, be_ref, w1c_ref, rb1_ref, wcp_ref, bcp_ref):
    wcp_ref[...] = jnp.dot(we_ref[...], w1c_ref[...],
                           preferred_element_type=jnp.float32)
    bcp_ref[...] = jnp.dot(be_ref[...], w1c_ref[...],
                           preferred_element_type=jnp.float32) + rb1_ref[...]


def _fold(we, be, w1c, rb1):
    return pl.pallas_call(
        _fold_body,
        out_shape=(jax.ShapeDtypeStruct((DE, H), jnp.float32),
                   jax.ShapeDtypeStruct((1, H), jnp.float32)),
    )(we, be, w1c, rb1)


def _head_body(g_ref, c_ref, w2_ref, b2_ref, w3_ref, b3_ref, o_ref):
    z = jnp.maximum(g_ref[...] + c_ref[...], 0.0)
    z = jnp.maximum(jnp.dot(z, w2_ref[...],
                            preferred_element_type=jnp.float32) + b2_ref[...],
                    0.0)
    s = jnp.dot(z, w3_ref[...], preferred_element_type=jnp.float32) + b3_ref[...]
    o_ref[...] = 1.0 / (1.0 + jnp.exp(-s))


def _head(g, c, w2p, b2p, w3p, b3p, bm):
    m = g.shape[0]
    return pl.pallas_call(
        _head_body,
        grid=(m // bm,),
        in_specs=[pl.BlockSpec((bm, H), lambda i: (i, 0)),
                  pl.BlockSpec((bm, H), lambda i: (i, 0)),
                  pl.BlockSpec((H, 32), lambda i: (0, 0)),
                  pl.BlockSpec((1, 32), lambda i: (0, 0)),
                  pl.BlockSpec((32, 8), lambda i: (0, 0)),
                  pl.BlockSpec((1, 8), lambda i: (0, 0))],
        out_specs=pl.BlockSpec((bm, 8), lambda i: (i, 0)),
        out_shape=jax.ShapeDtypeStruct((m, 8), jnp.float32),
    )(g, c, w2p, b2p, w3p, b3p)


@jax.jit
def kernel(x, edge_index, edge_attr, pos_edge_index, pos_edge_attr,
           neg_edge_index, neg_edge_attr, params):
    p = params
    h = _mm(x, p['node_W'], p['node_b'][None, :], 2000)
    ea = _mm(edge_attr, p['edge_W'], p['edge_b'][None, :], 8000)
    src = edge_index[0]
    dst = edge_index[1]

    w1a = p['r_W1'][0:H, :]
    w1b = p['r_W1'][H:2 * H, :]
    w1c = p['r_W1'][2 * H:3 * H, :]
    wcp, bcp = _fold(p['edge_W'], p['edge_b'][None, :], w1c, p['r_b1'][None, :])
    attr2 = jnp.concatenate([pos_edge_attr, neg_edge_attr], axis=0)
    cc = _mm(attr2, wcp, bcp, 8192)

    msg = _sc_message_kernel()
    for i in range(2):
        parts = msg(h, ea, src, dst)
        h = _layer(h, parts, p[f'conv{i}_W1'], p[f'conv{i}_b1'],
                   p[f'conv{i}_W2'], p[f'conv{i}_b2'],
                   p[f'bn{i}_g'], p[f'bn{i}_b'])

    a, b = _ab(h, w1a, w1b)
    ei2 = jnp.concatenate([pos_edge_index, neg_edge_index], axis=1)
    g = _sc_gather2_kernel()(a, b, ei2[0], ei2[1])

    w2p = jnp.pad(p['r_W2'], ((0, 0), (0, 7)))
    b2p = jnp.pad(p['r_b2'], (0, 7))[None, :]
    w3p = jnp.pad(p['r_W3'], ((0, 7), (0, 7)))
    b3p = jnp.pad(p['r_b3'], (0, 7))[None, :]
    out = _head(g, cc, w2p, b2p, w3p, b3p, 8192)
    return out[:EP, 0:1], out[EP:, 0:1], h


# speedup vs baseline: 4.3068x; 1.0398x over previous
import functools

import jax
import jax.numpy as jnp
from jax import lax
from jax.experimental import pallas as pl
from jax.experimental.pallas import tpu as pltpu
from jax.experimental.pallas import tpu_sc as plsc

N = 10000
E = 320000
EP = 65536
H = 128
DE = 16

NC, NS = 2, 16
NW = NC * NS
EW = E // NW
KE = 40
NCH = EW // KE
GROUPS = H // 16
ROWCH = 80
NZCH = N // ROWCH
ZITER = (NZCH + NS - 1) // NS

K2 = 128
EW2 = 2 * EP // NW
NCH2 = EW2 // K2


@functools.lru_cache(maxsize=None)
def _sc_mesh():
    return plsc.VectorSubcoreMesh(core_axis_name="c", subcore_axis_name="s",
                                  num_cores=NC, num_subcores=NS)


@functools.lru_cache(maxsize=None)
def _sc_message_kernel():
    @functools.partial(
        pl.kernel,
        out_type=jax.ShapeDtypeStruct((NC * N, H), jnp.float32),
        mesh=_sc_mesh(),
        scratch_types=[
            pltpu.VMEM((4, KE), jnp.int32),
            pltpu.VMEM((4, KE), jnp.int32),
            pltpu.VMEM((2, KE, H), jnp.float32),
            pltpu.VMEM((2, KE, H), jnp.float32),
            pltpu.VMEM_SHARED((N, H), jnp.float32),
            pltpu.SemaphoreType.DMA,
            pltpu.SemaphoreType.DMA,
            pltpu.SemaphoreType.DMA,
            pltpu.SemaphoreType.DMA,
            pltpu.SemaphoreType.DMA,
            pltpu.SemaphoreType.DMA,
            pltpu.SemaphoreType.DMA,
            pltpu.SemaphoreType.DMA,
        ],
    )
    def msg(h_hbm, ea_hbm, src_hbm, dst_hbm, out_hbm,
            src_v, dst_v, rows_v, ea_v, acc_sh,
            si0, si1, si2, si3, sg0, sg1, se0, se1):
        cid = lax.axis_index("c")
        sid = lax.axis_index("s")
        wid = sid * NC + cid
        sis = (si0, si1, si2, si3)
        sgs = (sg0, sg1)
        ses = (se0, se1)

        def zrow(r, _):
            for g in range(GROUPS):
                rows_v[0, r, pl.ds(g * 16, 16)] = jnp.zeros((16,), jnp.float32)
            return 0
        lax.fori_loop(0, KE, zrow, 0)

        def zchunk(t, _):
            c = sid + t * NS

            @pl.when(c < NZCH)
            def _():
                for rep in range(ROWCH // KE):
                    pltpu.sync_copy(
                        rows_v.at[0],
                        acc_sh.at[pl.ds(c * ROWCH + rep * KE, KE)])
            return 0
        lax.fori_loop(0, ZITER, zchunk, 0)
        plsc.subcore_barrier()

        def issue_idx(c, s):
            base = wid * EW + c * KE
            pltpu.async_copy(src_hbm.at[pl.ds(base, KE)], src_v.at[s], sis[s])
            pltpu.async_copy(dst_hbm.at[pl.ds(base, KE)], dst_v.at[s], sis[s])

        def wait_idx(s):
            pltpu.make_async_copy(src_hbm.at[pl.ds(0, KE)], src_v.at[s],
                                  sis[s]).wait()
            pltpu.make_async_copy(dst_hbm.at[pl.ds(0, KE)], dst_v.at[s],
                                  sis[s]).wait()

        def issue_data(c, i, d):
            base = wid * EW + c * KE
            pltpu.async_copy(h_hbm.at[src_v.at[i]], rows_v.at[d], sgs[d])
            pltpu.async_copy(ea_hbm.at[pl.ds(base, KE)], ea_v.at[d], ses[d])

        def wait_data(i, d):
            pltpu.make_async_copy(h_hbm.at[src_v.at[i]], rows_v.at[d],
                                  sgs[d]).wait()
            pltpu.make_async_copy(ea_hbm.at[pl.ds(0, KE)], ea_v.at[d],
                                  ses[d]).wait()

        def process(i, d):
            def rbody(r, _):
                for g in range(GROUPS):
                    sl = pl.ds(g * 16, 16)
                    rows_v[d, r, sl] = jnp.maximum(
                        rows_v[d, r, sl] + ea_v[d, r, sl], 0.0)
                return 0
            lax.fori_loop(0, KE, rbody, 0)
            pltpu.sync_copy(rows_v.at[d], acc_sh.at[dst_v.at[i]], add=True)

        NQ = NCH // 4
        PIPE = NQ * 4

        for s in range(4):
            pltpu.sync_copy(src_hbm.at[pl.ds(wid * EW + s * KE, KE)],
                            src_v.at[s])
            pltpu.sync_copy(dst_hbm.at[pl.ds(wid * EW + s * KE, KE)],
                            dst_v.at[s])
        issue_data(0, 0, 0)

        def quad(u, _):
            c0 = u * 4
            more = u < NQ - 1
            issue_data(c0 + 1, 1, 1)
            wait_data(0, 0)
            process(0, 0)

            @pl.when(more)
            def _():
                issue_idx(c0 + 4, 0)
            issue_data(c0 + 2, 2, 0)
            wait_data(1, 1)
            process(1, 1)

            @pl.when(more)
            def _():
                issue_idx(c0 + 5, 1)
            issue_data(c0 + 3, 3, 1)
            wait_data(2, 0)
            process(2, 0)

            @pl.when(more)
            def _():
                issue_idx(c0 + 6, 2)
                wait_idx(0)
                issue_data(c0 + 4, 0, 0)
            wait_data(3, 1)
            process(3, 1)

            @pl.when(more)
            def _():
                issue_idx(c0 + 7, 3)
                wait_idx(1)
                wait_idx(2)
                wait_idx(3)
            return 0
        lax.fori_loop(0, NQ, quad, 0)

        def tail(c, _):
            base = wid * EW + c * KE
            pltpu.sync_copy(src_hbm.at[pl.ds(base, KE)], src_v.at[0])
            pltpu.sync_copy(dst_hbm.at[pl.ds(base, KE)], dst_v.at[0])
            pltpu.async_copy(h_hbm.at[src_v.at[0]], rows_v.at[0], sg0).wait()
            pltpu.sync_copy(ea_hbm.at[pl.ds(base, KE)], ea_v.at[0])
            process(0, 0)
            return 0
        if PIPE < NCH:
            lax.fori_loop(PIPE, NCH, tail, 0)
        plsc.subcore_barrier()

        def wchunk(t, _):
            c = sid + t * NS

            @pl.when(c < NZCH)
            def _():
                off = cid * N + c * ROWCH
                for rep in range(ROWCH // KE):
                    pltpu.sync_copy(
                        acc_sh.at[pl.ds(c * ROWCH + rep * KE, KE)],
                        out_hbm.at[pl.ds(off + rep * KE, KE)])
            return 0
        lax.fori_loop(0, ZITER, wchunk, 0)
    return msg


@functools.lru_cache(maxsize=None)
def _sc_gather2_kernel():
    @functools.partial(
        pl.kernel,
        out_type=jax.ShapeDtypeStruct((2 * EP, H), jnp.float32),
        mesh=_sc_mesh(),
        scratch_types=[
            pltpu.VMEM((4, K2), jnp.int32),
            pltpu.VMEM((4, K2), jnp.int32),
            pltpu.VMEM((2, K2, H), jnp.float32),
            pltpu.VMEM((2, K2, H), jnp.float32),
            pltpu.SemaphoreType.DMA,
            pltpu.SemaphoreType.DMA,
            pltpu.SemaphoreType.DMA,
            pltpu.SemaphoreType.DMA,
            pltpu.SemaphoreType.DMA,
            pltpu.SemaphoreType.DMA,
            pltpu.SemaphoreType.DMA,
            pltpu.SemaphoreType.DMA,
        ],
    )
    def gat(a_hbm, b_hbm, s_hbm, d_hbm, out_hbm,
            si_v, di_v, ra_v, rb_v,
            xi0, xi1, xi2, xi3, sa0, sa1, sb0, sb1):
        cid = lax.axis_index("c")
        sid = lax.axis_index("s")
        wid = sid * NC + cid
        xis = (xi0, xi1, xi2, xi3)
        sas = (sa0, sa1)
        sbs = (sb0, sb1)

        def issue_idx(c, s):
            base = wid * EW2 + c * K2
            pltpu.async_copy(s_hbm.at[pl.ds(base, K2)], si_v.at[s], xis[s])
            pltpu.async_copy(d_hbm.at[pl.ds(base, K2)], di_v.at[s], xis[s])

        def wait_idx(s):
            pltpu.make_async_copy(s_hbm.at[pl.ds(0, K2)], si_v.at[s],
                                  xis[s]).wait()
            pltpu.make_async_copy(d_hbm.at[pl.ds(0, K2)], di_v.at[s],
                                  xis[s]).wait()

        def issue_data(i, d):
            pltpu.async_copy(a_hbm.at[si_v.at[i]], ra_v.at[d], sas[d])
            pltpu.async_copy(b_hbm.at[di_v.at[i]], rb_v.at[d], sbs[d])

        def wait_data(i, d):
            pltpu.make_async_copy(a_hbm.at[si_v.at[i]], ra_v.at[d],
                                  sas[d]).wait()
            pltpu.make_async_copy(b_hbm.at[di_v.at[i]], rb_v.at[d],
                                  sbs[d]).wait()

        def process(c, d):
            base = wid * EW2 + c * K2

            def rbody(r, _):
                for g in range(GROUPS):
                    sl = pl.ds(g * 16, 16)
                    ra_v[d, r, sl] = ra_v[d, r, sl] + rb_v[d, r, sl]
                return 0
            lax.fori_loop(0, K2, rbody, 0)
            pltpu.sync_copy(ra_v.at[d], out_hbm.at[pl.ds(base, K2)])

        NQ2 = NCH2 // 4
        for s in range(4):
            pltpu.sync_copy(s_hbm.at[pl.ds(wid * EW2 + s * K2, K2)],
                            si_v.at[s])
            pltpu.sync_copy(d_hbm.at[pl.ds(wid * EW2 + s * K2, K2)],
                            di_v.at[s])
        issue_data(0, 0)

        def quad(u, _):
            c0 = u * 4
            more = u < NQ2 - 1
            issue_data(1, 1)
            wait_data(0, 0)
            process(c0, 0)

            @pl.when(more)
            def _():
                issue_idx(c0 + 4, 0)
            issue_data(2, 0)
            wait_data(1, 1)
            process(c0 + 1, 1)

            @pl.when(more)
            def _():
                issue_idx(c0 + 5, 1)
            issue_data(3, 1)
            wait_data(2, 0)
            process(c0 + 2, 0)

            @pl.when(more)
            def _():
                issue_idx(c0 + 6, 2)
                wait_idx(0)
                issue_data(0, 0)
            wait_data(3, 1)
            process(c0 + 3, 1)

            @pl.when(more)
            def _():
                issue_idx(c0 + 7, 3)
                wait_idx(1)
                wait_idx(2)
                wait_idx(3)
            return 0
        lax.fori_loop(0, NQ2, quad, 0)
    return gat


def _mm_body(x_ref, w_ref, b_ref, o_ref):
    o_ref[...] = (jnp.dot(x_ref[...], w_ref[...],
                          preferred_element_type=jnp.float32) + b_ref[...])


def _mm(x, w, b, bm):
    m, k = x.shape
    n = w.shape[1]
    return pl.pallas_call(
        _mm_body,
        grid=(m // bm,),
        in_specs=[pl.BlockSpec((bm, k), lambda i: (i, 0)),
                  pl.BlockSpec((k, n), lambda i: (0, 0)),
                  pl.BlockSpec((1, n), lambda i: (0, 0))],
        out_specs=pl.BlockSpec((bm, n), lambda i: (i, 0)),
        out_shape=jax.ShapeDtypeStruct((m, n), jnp.float32),
    )(x, w, b)


def _layer_body(h_ref, p_ref, w1_ref, b1_ref, w2_ref, b2_ref, g_ref, bb_ref,
                o_ref):
    h = h_ref[...]
    z0 = h + p_ref[0:N, :] + p_ref[N:2 * N, :]
    z1 = jnp.maximum(jnp.dot(z0, w1_ref[...],
                             preferred_element_type=jnp.float32) + b1_ref[...],
                     0.0)
    z2 = jnp.dot(z1, w2_ref[...],
                 preferred_element_type=jnp.float32) + b2_ref[...]
    mu = jnp.mean(z2, axis=0, keepdims=True)
    var = jnp.mean((z2 - mu) ** 2, axis=0, keepdims=True)
    zn = (z2 - mu) / jnp.sqrt(var + 1e-5) * g_ref[...] + bb_ref[...]
    o_ref[...] = (h + jnp.maximum(zn, 0.0)) * 0.5


def _layer(h, parts, w1, b1, w2, b2, g, bb):
    return pl.pallas_call(
        _layer_body,
        out_shape=jax.ShapeDtypeStruct((N, H), jnp.float32),
    )(h, parts, w1, b1[None, :], w2, b2[None, :], g[None, :], bb[None, :])


def _ab_body(h_ref, wa_ref, wb_ref, a_ref, b_ref):
    hr = jnp.maximum(h_ref[...], 0.0)
    a_ref[...] = jnp.dot(hr, wa_ref[...], preferred_element_type=jnp.float32)
    b_ref[...] = jnp.dot(hr, wb_ref[...], preferred_element_type=jnp.float32)


def _ab(h, wa, wb):
    return pl.pallas_call(
        _ab_body,
        out_shape=(jax.ShapeDtypeStruct((N, H), jnp.float32),
                   jax.ShapeDtypeStruct((N, H), jnp.float32)),
    )(h, wa, wb)


def _fold_body(we_ref, be_ref, w1c_ref, rb1_ref, wcp_ref, bcp_ref):
    wcp_ref[...] = jnp.dot(we_ref[...], w1c_ref[...],
                           preferred_element_type=jnp.float32)
    bcp_ref[...] = jnp.dot(be_ref[...], w1c_ref[...],
                           preferred_element_type=jnp.float32) + rb1_ref[...]


def _fold(we, be, w1c, rb1):
    return pl.pallas_call(
        _fold_body,
        out_shape=(jax.ShapeDtypeStruct((DE, H), jnp.float32),
                   jax.ShapeDtypeStruct((1, H), jnp.float32)),
    )(we, be, w1c, rb1)


def _head_body(g_ref, c_ref, w2_ref, b2_ref, w3_ref, b3_ref, o_ref):
    z = jnp.maximum(g_ref[...] + c_ref[...], 0.0)
    z = jnp.maximum(jnp.dot(z, w2_ref[...],
                            preferred_element_type=jnp.float32) + b2_ref[...],
                    0.0)
    s = jnp.dot(z, w3_ref[...], preferred_element_type=jnp.float32) + b3_ref[...]
    o_ref[...] = 1.0 / (1.0 + jnp.exp(-s))


def _head(g, c, w2p, b2p, w3p, b3p, bm):
    m = g.shape[0]
    return pl.pallas_call(
        _head_body,
        grid=(m // bm,),
        in_specs=[pl.BlockSpec((bm, H), lambda i: (i, 0)),
                  pl.BlockSpec((bm, H), lambda i: (i, 0)),
                  pl.BlockSpec((H, 32), lambda i: (0, 0)),
                  pl.BlockSpec((1, 32), lambda i: (0, 0)),
                  pl.BlockSpec((32, 8), lambda i: (0, 0)),
                  pl.BlockSpec((1, 8), lambda i: (0, 0))],
        out_specs=pl.BlockSpec((bm, 8), lambda i: (i, 0)),
        out_shape=jax.ShapeDtypeStruct((m, 8), jnp.float32),
    )(g, c, w2p, b2p, w3p, b3p)


@jax.jit
def kernel(x, edge_index, edge_attr, pos_edge_index, pos_edge_attr,
           neg_edge_index, neg_edge_attr, params):
    p = params
    h = _mm(x, p['node_W'], p['node_b'][None, :], 2000)
    ea = _mm(edge_attr, p['edge_W'], p['edge_b'][None, :], 8000)
    src = edge_index[0]
    dst = edge_index[1]

    w1a = p['r_W1'][0:H, :]
    w1b = p['r_W1'][H:2 * H, :]
    w1c = p['r_W1'][2 * H:3 * H, :]
    wcp, bcp = _fold(p['edge_W'], p['edge_b'][None, :], w1c, p['r_b1'][None, :])
    attr2 = jnp.concatenate([pos_edge_attr, neg_edge_attr], axis=0)
    cc = _mm(attr2, wcp, bcp, 8192)

    msg = _sc_message_kernel()
    for i in range(2):
        parts = msg(h, ea, src, dst)
        h = _layer(h, parts, p[f'conv{i}_W1'], p[f'conv{i}_b1'],
                   p[f'conv{i}_W2'], p[f'conv{i}_b2'],
                   p[f'bn{i}_g'], p[f'bn{i}_b'])

    a, b = _ab(h, w1a, w1b)
    ei2 = jnp.concatenate([pos_edge_index, neg_edge_index], axis=1)
    g = _sc_gather2_kernel()(a, b, ei2[0], ei2[1])

    w2p = jnp.pad(p['r_W2'], ((0, 0), (0, 7)))
    b2p = jnp.pad(p['r_b2'], (0, 7))[None, :]
    w3p = jnp.pad(p['r_W3'], ((0, 7), (0, 7)))
    b3p = jnp.pad(p['r_b3'], (0, 7))[None, :]
    out = _head(g, cc, w2p, b2p, w3p, b3p, 8192)
    return out[:EP, 0:1], out[EP:, 0:1], h


# speedup vs baseline: 4.9189x; 1.1421x over previous
import functools

import jax
import jax.numpy as jnp
from jax import lax
from jax.experimental import pallas as pl
from jax.experimental.pallas import tpu as pltpu
from jax.experimental.pallas import tpu_sc as plsc

N = 10000
E = 320000
EP = 65536
H = 128
DE = 16

NC, NS = 2, 16
NW = NC * NS
EW = E // NW
KE = 80
NCH = EW // KE
GROUPS = H // 16
ROWCH = 80
NZCH = N // ROWCH
ZITER = (NZCH + NS - 1) // NS

K2 = 128
EW2 = 2 * EP // NW
NCH2 = EW2 // K2


@functools.lru_cache(maxsize=None)
def _sc_mesh():
    return plsc.VectorSubcoreMesh(core_axis_name="c", subcore_axis_name="s",
                                  num_cores=NC, num_subcores=NS)


@functools.lru_cache(maxsize=None)
def _sc_message_kernel():
    @functools.partial(
        pl.kernel,
        out_type=jax.ShapeDtypeStruct((NC * N, H), jnp.float32),
        mesh=_sc_mesh(),
        scratch_types=[
            pltpu.VMEM((4, KE), jnp.int32),
            pltpu.VMEM((4, KE), jnp.int32),
            pltpu.VMEM((2, KE, H), jnp.float32),
            pltpu.VMEM((2, KE, H), jnp.float32),
            pltpu.VMEM_SHARED((N, H), jnp.float32),
            pltpu.SemaphoreType.DMA,
            pltpu.SemaphoreType.DMA,
            pltpu.SemaphoreType.DMA,
            pltpu.SemaphoreType.DMA,
            pltpu.SemaphoreType.DMA,
            pltpu.SemaphoreType.DMA,
            pltpu.SemaphoreType.DMA,
            pltpu.SemaphoreType.DMA,
            pltpu.SemaphoreType.DMA,
            pltpu.SemaphoreType.DMA,
        ],
    )
    def msg(h_hbm, ea_hbm, src_hbm, dst_hbm, out_hbm,
            src_v, dst_v, rows_v, ea_v, acc_sh,
            si0, si1, si2, si3, sg0, sg1, se0, se1, sc0, sc1):
        cid = lax.axis_index("c")
        sid = lax.axis_index("s")
        wid = sid * NC + cid
        sis = (si0, si1, si2, si3)
        sgs = (sg0, sg1)
        ses = (se0, se1)
        scs = (sc0, sc1)

        def zrow(r, _):
            for g in range(GROUPS):
                rows_v[0, r, pl.ds(g * 16, 16)] = jnp.zeros((16,), jnp.float32)
            return 0
        lax.fori_loop(0, KE, zrow, 0)

        def zchunk(t, _):
            c = sid + t * NS

            @pl.when(c < NZCH)
            def _():
                for rep in range(ROWCH // KE):
                    pltpu.sync_copy(
                        rows_v.at[0],
                        acc_sh.at[pl.ds(c * ROWCH + rep * KE, KE)])
            return 0
        lax.fori_loop(0, ZITER, zchunk, 0)
        plsc.subcore_barrier()

        def issue_idx(c, s):
            base = wid * EW + c * KE
            pltpu.async_copy(src_hbm.at[pl.ds(base, KE)], src_v.at[s], sis[s])
            pltpu.async_copy(dst_hbm.at[pl.ds(base, KE)], dst_v.at[s], sis[s])

        def wait_idx(s):
            pltpu.make_async_copy(src_hbm.at[pl.ds(0, KE)], src_v.at[s],
                                  sis[s]).wait()
            pltpu.make_async_copy(dst_hbm.at[pl.ds(0, KE)], dst_v.at[s],
                                  sis[s]).wait()

        def issue_data(c, i, d):
            base = wid * EW + c * KE
            pltpu.async_copy(h_hbm.at[src_v.at[i]], rows_v.at[d], sgs[d])
            pltpu.async_copy(ea_hbm.at[pl.ds(base, KE)], ea_v.at[d], ses[d])

        def wait_data(i, d):
            pltpu.make_async_copy(h_hbm.at[src_v.at[i]], rows_v.at[d],
                                  sgs[d]).wait()
            pltpu.make_async_copy(ea_hbm.at[pl.ds(0, KE)], ea_v.at[d],
                                  ses[d]).wait()

        def process(i, d):
            def rbody(r, _):
                for g in range(GROUPS):
                    sl = pl.ds(g * 16, 16)
                    rows_v[d, r, sl] = jnp.maximum(
                        rows_v[d, r, sl] + ea_v[d, r, sl], 0.0)
                return 0
            lax.fori_loop(0, KE, rbody, 0)
            pltpu.sync_copy(rows_v.at[d], acc_sh.at[dst_v.at[i]], add=True)

        NQ = NCH // 4
        PIPE = NQ * 4

        for s in range(4):
            pltpu.sync_copy(src_hbm.at[pl.ds(wid * EW + s * KE, KE)],
                            src_v.at[s])
            pltpu.sync_copy(dst_hbm.at[pl.ds(wid * EW + s * KE, KE)],
                            dst_v.at[s])
        issue_data(0, 0, 0)

        def quad(u, _):
            c0 = u * 4
            more = u < NQ - 1
            issue_data(c0 + 1, 1, 1)
            wait_data(0, 0)
            process(0, 0)

            @pl.when(more)
            def _():
                issue_idx(c0 + 4, 0)
            issue_data(c0 + 2, 2, 0)
            wait_data(1, 1)
            process(1, 1)

            @pl.when(more)
            def _():
                issue_idx(c0 + 5, 1)
            issue_data(c0 + 3, 3, 1)
            wait_data(2, 0)
            process(2, 0)

            @pl.when(more)
            def _():
                issue_idx(c0 + 6, 2)
                wait_idx(0)
                issue_data(c0 + 4, 0, 0)
            wait_data(3, 1)
            process(3, 1)

            @pl.when(more)
            def _():
                issue_idx(c0 + 7, 3)
                wait_idx(1)
                wait_idx(2)
                wait_idx(3)
            return 0
        lax.fori_loop(0, NQ, quad, 0)

        def tail(c, _):
            base = wid * EW + c * KE
            pltpu.sync_copy(src_hbm.at[pl.ds(base, KE)], src_v.at[0])
            pltpu.sync_copy(dst_hbm.at[pl.ds(base, KE)], dst_v.at[0])
            pltpu.async_copy(h_hbm.at[src_v.at[0]], rows_v.at[0], sg0).wait()
            pltpu.sync_copy(ea_hbm.at[pl.ds(base, KE)], ea_v.at[0])
            process(0, 0)
            return 0
        if PIPE < NCH:
            lax.fori_loop(PIPE, NCH, tail, 0)
        plsc.subcore_barrier()

        def wchunk(t, _):
            c = sid + t * NS

            @pl.when(c < NZCH)
            def _():
                off = cid * N + c * ROWCH
                for rep in range(ROWCH // KE):
                    pltpu.sync_copy(
                        acc_sh.at[pl.ds(c * ROWCH + rep * KE, KE)],
                        out_hbm.at[pl.ds(off + rep * KE, KE)])
            return 0
        lax.fori_loop(0, ZITER, wchunk, 0)
    return msg


@functools.lru_cache(maxsize=None)
def _sc_gather2_kernel():
    @functools.partial(
        pl.kernel,
        out_type=jax.ShapeDtypeStruct((2 * EP, H), jnp.float32),
        mesh=_sc_mesh(),
        scratch_types=[
            pltpu.VMEM((4, K2), jnp.int32),
            pltpu.VMEM((4, K2), jnp.int32),
            pltpu.VMEM((2, K2, H), jnp.float32),
            pltpu.VMEM((2, K2, H), jnp.float32),
            pltpu.SemaphoreType.DMA,
            pltpu.SemaphoreType.DMA,
            pltpu.SemaphoreType.DMA,
            pltpu.SemaphoreType.DMA,
            pltpu.SemaphoreType.DMA,
            pltpu.SemaphoreType.DMA,
            pltpu.SemaphoreType.DMA,
            pltpu.SemaphoreType.DMA,
        ],
    )
    def gat(a_hbm, b_hbm, s_hbm, d_hbm, out_hbm,
            si_v, di_v, ra_v, rb_v,
            xi0, xi1, xi2, xi3, sa0, sa1, sb0, sb1):
        cid = lax.axis_index("c")
        sid = lax.axis_index("s")
        wid = sid * NC + cid
        xis = (xi0, xi1, xi2, xi3)
        sas = (sa0, sa1)
        sbs = (sb0, sb1)

        def issue_idx(c, s):
            base = wid * EW2 + c * K2
            pltpu.async_copy(s_hbm.at[pl.ds(base, K2)], si_v.at[s], xis[s])
            pltpu.async_copy(d_hbm.at[pl.ds(base, K2)], di_v.at[s], xis[s])

        def wait_idx(s):
            pltpu.make_async_copy(s_hbm.at[pl.ds(0, K2)], si_v.at[s],
                                  xis[s]).wait()
            pltpu.make_async_copy(d_hbm.at[pl.ds(0, K2)], di_v.at[s],
                                  xis[s]).wait()

        def issue_data(i, d):
            pltpu.async_copy(a_hbm.at[si_v.at[i]], ra_v.at[d], sas[d])
            pltpu.async_copy(b_hbm.at[di_v.at[i]], rb_v.at[d], sbs[d])

        def wait_data(i, d):
            pltpu.make_async_copy(a_hbm.at[si_v.at[i]], ra_v.at[d],
                                  sas[d]).wait()
            pltpu.make_async_copy(b_hbm.at[di_v.at[i]], rb_v.at[d],
                                  sbs[d]).wait()

        def process(c, d):
            base = wid * EW2 + c * K2

            def rbody(r, _):
                for g in range(GROUPS):
                    sl = pl.ds(g * 16, 16)
                    ra_v[d, r, sl] = ra_v[d, r, sl] + rb_v[d, r, sl]
                return 0
            lax.fori_loop(0, K2, rbody, 0)
            pltpu.sync_copy(ra_v.at[d], out_hbm.at[pl.ds(base, K2)])

        NQ2 = NCH2 // 4
        for s in range(4):
            pltpu.sync_copy(s_hbm.at[pl.ds(wid * EW2 + s * K2, K2)],
                            si_v.at[s])
            pltpu.sync_copy(d_hbm.at[pl.ds(wid * EW2 + s * K2, K2)],
                            di_v.at[s])
        issue_data(0, 0)

        def quad(u, _):
            c0 = u * 4
            more = u < NQ2 - 1
            issue_data(1, 1)
            wait_data(0, 0)
            process(c0, 0)

            @pl.when(more)
            def _():
                issue_idx(c0 + 4, 0)
            issue_data(2, 0)
            wait_data(1, 1)
            process(c0 + 1, 1)

            @pl.when(more)
            def _():
                issue_idx(c0 + 5, 1)
            issue_data(3, 1)
            wait_data(2, 0)
            process(c0 + 2, 0)

            @pl.when(more)
            def _():
                issue_idx(c0 + 6, 2)
                wait_idx(0)
                issue_data(0, 0)
            wait_data(3, 1)
            process(c0 + 3, 1)

            @pl.when(more)
            def _():
                issue_idx(c0 + 7, 3)
                wait_idx(1)
                wait_idx(2)
                wait_idx(3)
            return 0
        lax.fori_loop(0, NQ2, quad, 0)
    return gat


def _mm_body(x_ref, w_ref, b_ref, o_ref):
    o_ref[...] = (jnp.dot(x_ref[...], w_ref[...],
                          preferred_element_type=jnp.float32) + b_ref[...])


def _mm(x, w, b, bm):
    m, k = x.shape
    n = w.shape[1]
    return pl.pallas_call(
        _mm_body,
        grid=(m // bm,),
        in_specs=[pl.BlockSpec((bm, k), lambda i: (i, 0)),
                  pl.BlockSpec((k, n), lambda i: (0, 0)),
                  pl.BlockSpec((1, n), lambda i: (0, 0))],
        out_specs=pl.BlockSpec((bm, n), lambda i: (i, 0)),
        out_shape=jax.ShapeDtypeStruct((m, n), jnp.float32),
    )(x, w, b)


def _layer_body(h_ref, p_ref, w1_ref, b1_ref, w2_ref, b2_ref, g_ref, bb_ref,
                o_ref):
    h = h_ref[...]
    z0 = h + p_ref[0:N, :] + p_ref[N:2 * N, :]
    z1 = jnp.maximum(jnp.dot(z0, w1_ref[...],
                             preferred_element_type=jnp.float32) + b1_ref[...],
                     0.0)
    z2 = jnp.dot(z1, w2_ref[...],
                 preferred_element_type=jnp.float32) + b2_ref[...]
    mu = jnp.mean(z2, axis=0, keepdims=True)
    var = jnp.mean((z2 - mu) ** 2, axis=0, keepdims=True)
    zn = (z2 - mu) / jnp.sqrt(var + 1e-5) * g_ref[...] + bb_ref[...]
    o_ref[...] = (h + jnp.maximum(zn, 0.0)) * 0.5


def _layer(h, parts, w1, b1, w2, b2, g, bb):
    return pl.pallas_call(
        _layer_body,
        out_shape=jax.ShapeDtypeStruct((N, H), jnp.float32),
    )(h, parts, w1, b1[None, :], w2, b2[None, :], g[None, :], bb[None, :])


def _layer_ab_body(h_ref, p_ref, w1_ref, b1_ref, w2_ref, b2_ref, g_ref,
                   bb_ref, wa_ref, wb_ref, o_ref, a_ref, b_ref):
    h = h_ref[...]
    z0 = h + p_ref[0:N, :] + p_ref[N:2 * N, :]
    z1 = jnp.maximum(jnp.dot(z0, w1_ref[...],
                             preferred_element_type=jnp.float32) + b1_ref[...],
                     0.0)
    z2 = jnp.dot(z1, w2_ref[...],
                 preferred_element_type=jnp.float32) + b2_ref[...]
    mu = jnp.mean(z2, axis=0, keepdims=True)
    var = jnp.mean((z2 - mu) ** 2, axis=0, keepdims=True)
    zn = (z2 - mu) / jnp.sqrt(var + 1e-5) * g_ref[...] + bb_ref[...]
    hn = (h + jnp.maximum(zn, 0.0)) * 0.5
    o_ref[...] = hn
    hr = jnp.maximum(hn, 0.0)
    a_ref[...] = jnp.dot(hr, wa_ref[...], preferred_element_type=jnp.float32)
    b_ref[...] = jnp.dot(hr, wb_ref[...], preferred_element_type=jnp.float32)


def _layer_ab(h, parts, w1, b1, w2, b2, g, bb, wa, wb):
    return pl.pallas_call(
        _layer_ab_body,
        out_shape=(jax.ShapeDtypeStruct((N, H), jnp.float32),
                   jax.ShapeDtypeStruct((N, H), jnp.float32),
                   jax.ShapeDtypeStruct((N, H), jnp.float32)),
    )(h, parts, w1, b1[None, :], w2, b2[None, :], g[None, :], bb[None, :],
      wa, wb)


def _head_body(g_ref, at_ref, we_ref, be_ref, w1c_ref, rb1_ref,
               w2_ref, b2_ref, w3_ref, b3_ref, o_ref):
    wcp = jnp.dot(we_ref[...], w1c_ref[...],
                  preferred_element_type=jnp.float32)
    bcp = jnp.dot(be_ref[...], w1c_ref[...],
                  preferred_element_type=jnp.float32) + rb1_ref[...]
    c = jnp.dot(at_ref[...], wcp, preferred_element_type=jnp.float32) + bcp
    z = jnp.maximum(g_ref[...] + c, 0.0)
    z = jnp.maximum(jnp.dot(z, w2_ref[...],
                            preferred_element_type=jnp.float32) + b2_ref[...],
                    0.0)
    s = jnp.dot(z, w3_ref[...], preferred_element_type=jnp.float32) + b3_ref[...]
    o_ref[...] = 1.0 / (1.0 + jnp.exp(-s))


def _head(g, attr2, we, be, w1c, rb1, w2p, b2p, w3p, b3p, bm):
    m = g.shape[0]
    return pl.pallas_call(
        _head_body,
        grid=(m // bm,),
        in_specs=[pl.BlockSpec((bm, H), lambda i: (i, 0)),
                  pl.BlockSpec((bm, DE), lambda i: (i, 0)),
                  pl.BlockSpec((DE, H), lambda i: (0, 0)),
                  pl.BlockSpec((1, H), lambda i: (0, 0)),
                  pl.BlockSpec((H, H), lambda i: (0, 0)),
                  pl.BlockSpec((1, H), lambda i: (0, 0)),
                  pl.BlockSpec((H, 32), lambda i: (0, 0)),
                  pl.BlockSpec((1, 32), lambda i: (0, 0)),
                  pl.BlockSpec((32, 8), lambda i: (0, 0)),
                  pl.BlockSpec((1, 8), lambda i: (0, 0))],
        out_specs=pl.BlockSpec((bm, 8), lambda i: (i, 0)),
        out_shape=jax.ShapeDtypeStruct((m, 8), jnp.float32),
    )(g, attr2, we, be, w1c, rb1, w2p, b2p, w3p, b3p)


@jax.jit
def kernel(x, edge_index, edge_attr, pos_edge_index, pos_edge_attr,
           neg_edge_index, neg_edge_attr, params):
    p = params
    h = _mm(x, p['node_W'], p['node_b'][None, :], 2000)
    ea = _mm(edge_attr, p['edge_W'], p['edge_b'][None, :], 8000)
    src = edge_index[0]
    dst = edge_index[1]

    w1a = p['r_W1'][0:H, :]
    w1b = p['r_W1'][H:2 * H, :]
    w1c = p['r_W1'][2 * H:3 * H, :]
    attr2 = jnp.concatenate([pos_edge_attr, neg_edge_attr], axis=0)

    msg = _sc_message_kernel()
    parts = msg(h, ea, src, dst)
    h = _layer(h, parts, p['conv0_W1'], p['conv0_b1'],
               p['conv0_W2'], p['conv0_b2'], p['bn0_g'], p['bn0_b'])
    parts = msg(h, ea, src, dst)
    h, a, b = _layer_ab(h, parts, p['conv1_W1'], p['conv1_b1'],
                        p['conv1_W2'], p['conv1_b2'], p['bn1_g'], p['bn1_b'],
                        w1a, w1b)

    ei2 = jnp.concatenate([pos_edge_index, neg_edge_index], axis=1)
    g = _sc_gather2_kernel()(a, b, ei2[0], ei2[1])

    w2p = jnp.pad(p['r_W2'], ((0, 0), (0, 7)))
    b2p = jnp.pad(p['r_b2'], (0, 7))[None, :]
    w3p = jnp.pad(p['r_W3'], ((0, 7), (0, 7)))
    b3p = jnp.pad(p['r_b3'], (0, 7))[None, :]
    out = _head(g, attr2, p['edge_W'], p['edge_b'][None, :], w1c,
                p['r_b1'][None, :], w2p, b2p, w3p, b3p, 8192)
    return out[:EP, 0:1], out[EP:, 0:1], h


# speedup vs baseline: 4.9851x; 1.0135x over previous
import functools

import jax
import jax.numpy as jnp
from jax import lax
from jax.experimental import pallas as pl
from jax.experimental.pallas import tpu as pltpu
from jax.experimental.pallas import tpu_sc as plsc

N = 10000
E = 320000
EP = 65536
H = 128
DE = 16

NC, NS = 2, 16
NW = NC * NS
EW = E // NW
KE = 40
NCH = EW // KE
GROUPS = H // 16
ROWCH = 80
NZCH = N // ROWCH
ZITER = (NZCH + NS - 1) // NS

K2 = 128
EW2 = 2 * EP // NW
NCH2 = EW2 // K2


@functools.lru_cache(maxsize=None)
def _sc_mesh():
    return plsc.VectorSubcoreMesh(core_axis_name="c", subcore_axis_name="s",
                                  num_cores=NC, num_subcores=NS)


@functools.lru_cache(maxsize=None)
def _sc_message_kernel():
    @functools.partial(
        pl.kernel,
        out_type=jax.ShapeDtypeStruct((NC * N, H), jnp.float32),
        mesh=_sc_mesh(),
        scratch_types=[
            pltpu.VMEM((8, KE), jnp.int32),
            pltpu.VMEM((8, KE), jnp.int32),
            pltpu.VMEM((4, KE, H), jnp.float32),
            pltpu.VMEM((4, KE, H), jnp.float32),
            pltpu.VMEM_SHARED((N, H), jnp.float32),
        ] + [pltpu.SemaphoreType.DMA] * 20,
    )
    def msg(h_hbm, ea_hbm, src_hbm, dst_hbm, out_hbm,
            src_v, dst_v, rows_v, ea_v, acc_sh, *sems):
        cid = lax.axis_index("c")
        sid = lax.axis_index("s")
        wid = sid * NC + cid
        sis = sems[0:8]
        sgs = sems[8:12]
        ses = sems[12:16]
        scs = sems[16:20]

        def zrow(r, _):
            for g in range(GROUPS):
                rows_v[0, r, pl.ds(g * 16, 16)] = jnp.zeros((16,), jnp.float32)
            return 0
        lax.fori_loop(0, KE, zrow, 0)

        def zchunk(t, _):
            c = sid + t * NS

            @pl.when(c < NZCH)
            def _():
                for rep in range(ROWCH // KE):
                    pltpu.sync_copy(
                        rows_v.at[0],
                        acc_sh.at[pl.ds(c * ROWCH + rep * KE, KE)])
            return 0
        lax.fori_loop(0, ZITER, zchunk, 0)
        plsc.subcore_barrier()

        def issue_idx(c, s):
            base = wid * EW + c * KE
            pltpu.async_copy(src_hbm.at[pl.ds(base, KE)], src_v.at[s], sis[s])
            pltpu.async_copy(dst_hbm.at[pl.ds(base, KE)], dst_v.at[s], sis[s])

        def wait_idx(s):
            pltpu.make_async_copy(src_hbm.at[pl.ds(0, KE)], src_v.at[s],
                                  sis[s]).wait()
            pltpu.make_async_copy(dst_hbm.at[pl.ds(0, KE)], dst_v.at[s],
                                  sis[s]).wait()

        def issue_data(c, si, d):
            base = wid * EW + c * KE
            pltpu.async_copy(h_hbm.at[src_v.at[si]], rows_v.at[d], sgs[d])
            pltpu.async_copy(ea_hbm.at[pl.ds(base, KE)], ea_v.at[d], ses[d])

        def wait_data(si, d):
            pltpu.make_async_copy(h_hbm.at[src_v.at[si]], rows_v.at[d],
                                  sgs[d]).wait()
            pltpu.make_async_copy(ea_hbm.at[pl.ds(0, KE)], ea_v.at[d],
                                  ses[d]).wait()

        def process(si, d):
            def rbody(r, _):
                for g in range(GROUPS):
                    sl = pl.ds(g * 16, 16)
                    rows_v[d, r, sl] = jnp.maximum(
                        rows_v[d, r, sl] + ea_v[d, r, sl], 0.0)
                return 0
            lax.fori_loop(0, KE, rbody, 0)
            pltpu.async_copy(rows_v.at[d], acc_sh.at[dst_v.at[si]], scs[d],
                             add=True)

        def wait_scatter(si, d):
            pltpu.make_async_copy(rows_v.at[d], acc_sh.at[dst_v.at[si]],
                                  scs[d]).wait()

        def steady(c, si, d, first=False):
            wait_data(si, d)
            process(si, d)
            if not first:
                wait_scatter((si + 6) % 8, (d + 2) % 4)
            issue_idx(c + 6, (si + 6) % 8)
            wait_idx((si + 2) % 8)
            issue_data(c + 2, (si + 2) % 8, (d + 2) % 4)

        pltpu.sync_copy(src_hbm.at[pl.ds(wid * EW, KE)], src_v.at[0])
        pltpu.sync_copy(dst_hbm.at[pl.ds(wid * EW, KE)], dst_v.at[0])
        pltpu.sync_copy(src_hbm.at[pl.ds(wid * EW + KE, KE)], src_v.at[1])
        pltpu.sync_copy(dst_hbm.at[pl.ds(wid * EW + KE, KE)], dst_v.at[1])
        for s in range(2, 6):
            issue_idx(s, s)
        issue_data(0, 0, 0)
        issue_data(1, 1, 1)

        for c in range(2):
            steady(c, c, c, first=True)
        for c in range(2, 8):
            steady(c, c, c % 4)

        def octad(u, _):
            c0 = u * 8
            for j in range(8):
                steady(c0 + j, j, j % 4)
            return 0
        NSTEADY = (NCH - 6 - 8) // 8
        lax.fori_loop(1, 1 + NSTEADY, octad, 0)

        TAILC = 8 + NSTEADY * 8
        for c in range(TAILC, NCH):
            si = c % 8
            d = c % 4
            wait_data(si, d)
            process(si, d)
            wait_scatter((si + 6) % 8, (d + 2) % 4)
            if c + 6 < NCH:
                issue_idx(c + 6, (si + 6) % 8)
            if c + 2 < NCH:
                wait_idx((si + 2) % 8)
                issue_data(c + 2, (si + 2) % 8, (d + 2) % 4)
        wait_scatter((NCH - 2) % 8, (NCH - 2) % 4)
        wait_scatter((NCH - 1) % 8, (NCH - 1) % 4)
        plsc.subcore_barrier()

        def wchunk(t, _):
            c = sid + t * NS

            @pl.when(c < NZCH)
            def _():
                off = cid * N + c * ROWCH
                for rep in range(ROWCH // KE):
                    pltpu.sync_copy(
                        acc_sh.at[pl.ds(c * ROWCH + rep * KE, KE)],
                        out_hbm.at[pl.ds(off + rep * KE, KE)])
            return 0
        lax.fori_loop(0, ZITER, wchunk, 0)
    return msg


@functools.lru_cache(maxsize=None)
def _sc_gather2_kernel():
    @functools.partial(
        pl.kernel,
        out_type=jax.ShapeDtypeStruct((2 * EP, H), jnp.float32),
        mesh=_sc_mesh(),
        scratch_types=[
            pltpu.VMEM((4, K2), jnp.int32),
            pltpu.VMEM((4, K2), jnp.int32),
            pltpu.VMEM((2, K2, H), jnp.float32),
            pltpu.VMEM((2, K2, H), jnp.float32),
            pltpu.SemaphoreType.DMA,
            pltpu.SemaphoreType.DMA,
            pltpu.SemaphoreType.DMA,
            pltpu.SemaphoreType.DMA,
            pltpu.SemaphoreType.DMA,
            pltpu.SemaphoreType.DMA,
            pltpu.SemaphoreType.DMA,
            pltpu.SemaphoreType.DMA,
        ],
    )
    def gat(a_hbm, b_hbm, s_hbm, d_hbm, out_hbm,
            si_v, di_v, ra_v, rb_v,
            xi0, xi1, xi2, xi3, sa0, sa1, sb0, sb1):
        cid = lax.axis_index("c")
        sid = lax.axis_index("s")
        wid = sid * NC + cid
        xis = (xi0, xi1, xi2, xi3)
        sas = (sa0, sa1)
        sbs = (sb0, sb1)

        def issue_idx(c, s):
            base = wid * EW2 + c * K2
            pltpu.async_copy(s_hbm.at[pl.ds(base, K2)], si_v.at[s], xis[s])
            pltpu.async_copy(d_hbm.at[pl.ds(base, K2)], di_v.at[s], xis[s])

        def wait_idx(s):
            pltpu.make_async_copy(s_hbm.at[pl.ds(0, K2)], si_v.at[s],
                                  xis[s]).wait()
            pltpu.make_async_copy(d_hbm.at[pl.ds(0, K2)], di_v.at[s],
                                  xis[s]).wait()

        def issue_data(i, d):
            pltpu.async_copy(a_hbm.at[si_v.at[i]], ra_v.at[d], sas[d])
            pltpu.async_copy(b_hbm.at[di_v.at[i]], rb_v.at[d], sbs[d])

        def wait_data(i, d):
            pltpu.make_async_copy(a_hbm.at[si_v.at[i]], ra_v.at[d],
                                  sas[d]).wait()
            pltpu.make_async_copy(b_hbm.at[di_v.at[i]], rb_v.at[d],
                                  sbs[d]).wait()

        def process(c, d):
            base = wid * EW2 + c * K2

            def rbody(r, _):
                for g in range(GROUPS):
                    sl = pl.ds(g * 16, 16)
                    ra_v[d, r, sl] = ra_v[d, r, sl] + rb_v[d, r, sl]
                return 0
            lax.fori_loop(0, K2, rbody, 0)
            pltpu.sync_copy(ra_v.at[d], out_hbm.at[pl.ds(base, K2)])

        NQ2 = NCH2 // 4
        for s in range(4):
            pltpu.sync_copy(s_hbm.at[pl.ds(wid * EW2 + s * K2, K2)],
                            si_v.at[s])
            pltpu.sync_copy(d_hbm.at[pl.ds(wid * EW2 + s * K2, K2)],
                            di_v.at[s])
        issue_data(0, 0)

        def quad(u, _):
            c0 = u * 4
            more = u < NQ2 - 1
            issue_data(1, 1)
            wait_data(0, 0)
            process(c0, 0)

            @pl.when(more)
            def _():
                issue_idx(c0 + 4, 0)
            issue_data(2, 0)
            wait_data(1, 1)
            process(c0 + 1, 1)

            @pl.when(more)
            def _():
                issue_idx(c0 + 5, 1)
            issue_data(3, 1)
            wait_data(2, 0)
            process(c0 + 2, 0)

            @pl.when(more)
            def _():
                issue_idx(c0 + 6, 2)
                wait_idx(0)
                issue_data(0, 0)
            wait_data(3, 1)
            process(c0 + 3, 1)

            @pl.when(more)
            def _():
                issue_idx(c0 + 7, 3)
                wait_idx(1)
                wait_idx(2)
                wait_idx(3)
            return 0
        lax.fori_loop(0, NQ2, quad, 0)
    return gat


def _mm_body(x_ref, w_ref, b_ref, o_ref):
    o_ref[...] = (jnp.dot(x_ref[...], w_ref[...],
                          preferred_element_type=jnp.float32) + b_ref[...])


def _mm(x, w, b, bm):
    m, k = x.shape
    n = w.shape[1]
    return pl.pallas_call(
        _mm_body,
        grid=(m // bm,),
        in_specs=[pl.BlockSpec((bm, k), lambda i: (i, 0)),
                  pl.BlockSpec((k, n), lambda i: (0, 0)),
                  pl.BlockSpec((1, n), lambda i: (0, 0))],
        out_specs=pl.BlockSpec((bm, n), lambda i: (i, 0)),
        out_shape=jax.ShapeDtypeStruct((m, n), jnp.float32),
    )(x, w, b)


def _layer_body(h_ref, p_ref, w1_ref, b1_ref, w2_ref, b2_ref, g_ref, bb_ref,
                o_ref):
    h = h_ref[...]
    z0 = h + p_ref[0:N, :] + p_ref[N:2 * N, :]
    z1 = jnp.maximum(jnp.dot(z0, w1_ref[...],
                             preferred_element_type=jnp.float32) + b1_ref[...],
                     0.0)
    z2 = jnp.dot(z1, w2_ref[...],
                 preferred_element_type=jnp.float32) + b2_ref[...]
    mu = jnp.mean(z2, axis=0, keepdims=True)
    var = jnp.mean((z2 - mu) ** 2, axis=0, keepdims=True)
    zn = (z2 - mu) / jnp.sqrt(var + 1e-5) * g_ref[...] + bb_ref[...]
    o_ref[...] = (h + jnp.maximum(zn, 0.0)) * 0.5


def _layer(h, parts, w1, b1, w2, b2, g, bb):
    return pl.pallas_call(
        _layer_body,
        out_shape=jax.ShapeDtypeStruct((N, H), jnp.float32),
    )(h, parts, w1, b1[None, :], w2, b2[None, :], g[None, :], bb[None, :])


def _layer_ab_body(h_ref, p_ref, w1_ref, b1_ref, w2_ref, b2_ref, g_ref,
                   bb_ref, wa_ref, wb_ref, o_ref, a_ref, b_ref):
    h = h_ref[...]
    z0 = h + p_ref[0:N, :] + p_ref[N:2 * N, :]
    z1 = jnp.maximum(jnp.dot(z0, w1_ref[...],
                             preferred_element_type=jnp.float32) + b1_ref[...],
                     0.0)
    z2 = jnp.dot(z1, w2_ref[...],
                 preferred_element_type=jnp.float32) + b2_ref[...]
    mu = jnp.mean(z2, axis=0, keepdims=True)
    var = jnp.mean((z2 - mu) ** 2, axis=0, keepdims=True)
    zn = (z2 - mu) / jnp.sqrt(var + 1e-5) * g_ref[...] + bb_ref[...]
    hn = (h + jnp.maximum(zn, 0.0)) * 0.5
    o_ref[...] = hn
    hr = jnp.maximum(hn, 0.0)
    a_ref[...] = jnp.dot(hr, wa_ref[...], preferred_element_type=jnp.float32)
    b_ref[...] = jnp.dot(hr, wb_ref[...], preferred_element_type=jnp.float32)


def _layer_ab(h, parts, w1, b1, w2, b2, g, bb, wa, wb):
    return pl.pallas_call(
        _layer_ab_body,
        out_shape=(jax.ShapeDtypeStruct((N, H), jnp.float32),
                   jax.ShapeDtypeStruct((N, H), jnp.float32),
                   jax.ShapeDtypeStruct((N, H), jnp.float32)),
    )(h, parts, w1, b1[None, :], w2, b2[None, :], g[None, :], bb[None, :],
      wa, wb)


def _head_body(g_ref, at_ref, we_ref, be_ref, w1c_ref, rb1_ref,
               w2_ref, b2_ref, w3_ref, b3_ref, o_ref):
    wcp = jnp.dot(we_ref[...], w1c_ref[...],
                  preferred_element_type=jnp.float32)
    bcp = jnp.dot(be_ref[...], w1c_ref[...],
                  preferred_element_type=jnp.float32) + rb1_ref[...]
    c = jnp.dot(at_ref[...], wcp, preferred_element_type=jnp.float32) + bcp
    z = jnp.maximum(g_ref[...] + c, 0.0)
    z = jnp.maximum(jnp.dot(z, w2_ref[...],
                            preferred_element_type=jnp.float32) + b2_ref[...],
                    0.0)
    s = jnp.dot(z, w3_ref[...], preferred_element_type=jnp.float32) + b3_ref[...]
    o_ref[...] = 1.0 / (1.0 + jnp.exp(-s))


def _head(g, attr2, we, be, w1c, rb1, w2p, b2p, w3p, b3p, bm):
    m = g.shape[0]
    return pl.pallas_call(
        _head_body,
        grid=(m // bm,),
        in_specs=[pl.BlockSpec((bm, H), lambda i: (i, 0)),
                  pl.BlockSpec((bm, DE), lambda i: (i, 0)),
                  pl.BlockSpec((DE, H), lambda i: (0, 0)),
                  pl.BlockSpec((1, H), lambda i: (0, 0)),
                  pl.BlockSpec((H, H), lambda i: (0, 0)),
                  pl.BlockSpec((1, H), lambda i: (0, 0)),
                  pl.BlockSpec((H, 32), lambda i: (0, 0)),
                  pl.BlockSpec((1, 32), lambda i: (0, 0)),
                  pl.BlockSpec((32, 8), lambda i: (0, 0)),
                  pl.BlockSpec((1, 8), lambda i: (0, 0))],
        out_specs=pl.BlockSpec((bm, 8), lambda i: (i, 0)),
        out_shape=jax.ShapeDtypeStruct((m, 8), jnp.float32),
    )(g, attr2, we, be, w1c, rb1, w2p, b2p, w3p, b3p)


@jax.jit
def kernel(x, edge_index, edge_attr, pos_edge_index, pos_edge_attr,
           neg_edge_index, neg_edge_attr, params):
    p = params
    h = _mm(x, p['node_W'], p['node_b'][None, :], 2000)
    ea = _mm(edge_attr, p['edge_W'], p['edge_b'][None, :], 8000)
    src = edge_index[0]
    dst = edge_index[1]

    w1a = p['r_W1'][0:H, :]
    w1b = p['r_W1'][H:2 * H, :]
    w1c = p['r_W1'][2 * H:3 * H, :]
    attr2 = jnp.concatenate([pos_edge_attr, neg_edge_attr], axis=0)

    msg = _sc_message_kernel()
    parts = msg(h, ea, src, dst)
    h = _layer(h, parts, p['conv0_W1'], p['conv0_b1'],
               p['conv0_W2'], p['conv0_b2'], p['bn0_g'], p['bn0_b'])
    parts = msg(h, ea, src, dst)
    h, a, b = _layer_ab(h, parts, p['conv1_W1'], p['conv1_b1'],
                        p['conv1_W2'], p['conv1_b2'], p['bn1_g'], p['bn1_b'],
                        w1a, w1b)

    ei2 = jnp.concatenate([pos_edge_index, neg_edge_index], axis=1)
    g = _sc_gather2_kernel()(a, b, ei2[0], ei2[1])

    w2p = jnp.pad(p['r_W2'], ((0, 0), (0, 7)))
    b2p = jnp.pad(p['r_b2'], (0, 7))[None, :]
    w3p = jnp.pad(p['r_W3'], ((0, 7), (0, 7)))
    b3p = jnp.pad(p['r_b3'], (0, 7))[None, :]
    out = _head(g, attr2, p['edge_W'], p['edge_b'][None, :], w1c,
                p['r_b1'][None, :], w2p, b2p, w3p, b3p, 8192)
    return out[:EP, 0:1], out[EP:, 0:1], h


# speedup vs baseline: 5.0187x; 1.0067x over previous
import functools

import jax
import jax.numpy as jnp
from jax import lax
from jax.experimental import pallas as pl
from jax.experimental.pallas import tpu as pltpu
from jax.experimental.pallas import tpu_sc as plsc

N = 10000
E = 320000
EP = 65536
H = 128
DE = 16

NC, NS = 2, 16
NW = NC * NS
EW = E // NW
KE = 40
NCH = EW // KE
GROUPS = H // 16
ROWCH = 80
NZCH = N // ROWCH
ZITER = (NZCH + NS - 1) // NS

K2 = 128
EW2 = 2 * EP // NW
NCH2 = EW2 // K2


@functools.lru_cache(maxsize=None)
def _sc_mesh():
    return plsc.VectorSubcoreMesh(core_axis_name="c", subcore_axis_name="s",
                                  num_cores=NC, num_subcores=NS)


@functools.lru_cache(maxsize=None)
def _sc_message_kernel():
    @functools.partial(
        pl.kernel,
        out_type=jax.ShapeDtypeStruct((NC * N, H), jnp.float32),
        mesh=_sc_mesh(),
        scratch_types=[
            pltpu.VMEM((8, KE), jnp.int32),
            pltpu.VMEM((8, KE), jnp.int32),
            pltpu.VMEM((4, KE, H), jnp.float32),
            pltpu.VMEM((4, KE, H), jnp.float32),
            pltpu.VMEM_SHARED((N, H), jnp.float32),
        ] + [pltpu.SemaphoreType.DMA] * 20,
    )
    def msg(h_hbm, ea_hbm, src_hbm, dst_hbm, out_hbm,
            src_v, dst_v, rows_v, ea_v, acc_sh, *sems):
        cid = lax.axis_index("c")
        sid = lax.axis_index("s")
        wid = sid * NC + cid
        sis = sems[0:8]
        sgs = sems[8:12]
        ses = sems[12:16]
        scs = sems[16:20]

        def zrow(r, _):
            for g in range(GROUPS):
                rows_v[0, r, pl.ds(g * 16, 16)] = jnp.zeros((16,), jnp.float32)
            return 0
        lax.fori_loop(0, KE, zrow, 0)

        def zchunk(t, _):
            c = sid + t * NS

            @pl.when(c < NZCH)
            def _():
                for rep in range(ROWCH // KE):
                    pltpu.sync_copy(
                        rows_v.at[0],
                        acc_sh.at[pl.ds(c * ROWCH + rep * KE, KE)])
            return 0
        lax.fori_loop(0, ZITER, zchunk, 0)
        plsc.subcore_barrier()

        def issue_idx(c, s):
            base = wid * EW + c * KE
            pltpu.async_copy(src_hbm.at[pl.ds(base, KE)], src_v.at[s], sis[s])
            pltpu.async_copy(dst_hbm.at[pl.ds(base, KE)], dst_v.at[s], sis[s])

        def wait_idx(s):
            pltpu.make_async_copy(src_hbm.at[pl.ds(0, KE)], src_v.at[s],
                                  sis[s]).wait()
            pltpu.make_async_copy(dst_hbm.at[pl.ds(0, KE)], dst_v.at[s],
                                  sis[s]).wait()

        def issue_data(c, si, d):
            base = wid * EW + c * KE
            pltpu.async_copy(h_hbm.at[src_v.at[si]], rows_v.at[d], sgs[d])
            pltpu.async_copy(ea_hbm.at[pl.ds(base, KE)], ea_v.at[d], ses[d])

        def wait_data(si, d):
            pltpu.make_async_copy(h_hbm.at[src_v.at[si]], rows_v.at[d],
                                  sgs[d]).wait()
            pltpu.make_async_copy(ea_hbm.at[pl.ds(0, KE)], ea_v.at[d],
                                  ses[d]).wait()

        def process(si, d):
            def rbody(r, _):
                for g in range(GROUPS):
                    sl = pl.ds(g * 16, 16)
                    rows_v[d, r, sl] = jnp.maximum(
                        rows_v[d, r, sl] + ea_v[d, r, sl], 0.0)
                return 0
            lax.fori_loop(0, KE, rbody, 0)
            pltpu.async_copy(rows_v.at[d], acc_sh.at[dst_v.at[si]], scs[d],
                             add=True)

        def wait_scatter(si, d):
            pltpu.make_async_copy(rows_v.at[d], acc_sh.at[dst_v.at[si]],
                                  scs[d]).wait()

        def steady(c, si, d, first=False):
            wait_data(si, d)
            process(si, d)
            if not first:
                wait_scatter((si + 6) % 8, (d + 2) % 4)
            issue_idx(c + 6, (si + 6) % 8)
            wait_idx((si + 2) % 8)
            issue_data(c + 2, (si + 2) % 8, (d + 2) % 4)

        pltpu.sync_copy(src_hbm.at[pl.ds(wid * EW, KE)], src_v.at[0])
        pltpu.sync_copy(dst_hbm.at[pl.ds(wid * EW, KE)], dst_v.at[0])
        pltpu.sync_copy(src_hbm.at[pl.ds(wid * EW + KE, KE)], src_v.at[1])
        pltpu.sync_copy(dst_hbm.at[pl.ds(wid * EW + KE, KE)], dst_v.at[1])
        for s in range(2, 6):
            issue_idx(s, s)
        issue_data(0, 0, 0)
        issue_data(1, 1, 1)

        for c in range(2):
            steady(c, c, c, first=True)
        for c in range(2, 8):
            steady(c, c, c % 4)

        def octad(u, _):
            c0 = u * 8
            for j in range(8):
                steady(c0 + j, j, j % 4)
            return 0
        NSTEADY = (NCH - 6 - 8) // 8
        lax.fori_loop(1, 1 + NSTEADY, octad, 0)

        TAILC = 8 + NSTEADY * 8
        for c in range(TAILC, NCH):
            si = c % 8
            d = c % 4
            wait_data(si, d)
            process(si, d)
            wait_scatter((si + 6) % 8, (d + 2) % 4)
            if c + 6 < NCH:
                issue_idx(c + 6, (si + 6) % 8)
            if c + 2 < NCH:
                wait_idx((si + 2) % 8)
                issue_data(c + 2, (si + 2) % 8, (d + 2) % 4)
        wait_scatter((NCH - 2) % 8, (NCH - 2) % 4)
        wait_scatter((NCH - 1) % 8, (NCH - 1) % 4)
        plsc.subcore_barrier()

        def wchunk(t, _):
            c = sid + t * NS

            @pl.when(c < NZCH)
            def _():
                off = cid * N + c * ROWCH
                for rep in range(ROWCH // KE):
                    pltpu.sync_copy(
                        acc_sh.at[pl.ds(c * ROWCH + rep * KE, KE)],
                        out_hbm.at[pl.ds(off + rep * KE, KE)])
            return 0
        lax.fori_loop(0, ZITER, wchunk, 0)
    return msg


@functools.lru_cache(maxsize=None)
def _sc_gather2_kernel():
    @functools.partial(
        pl.kernel,
        out_type=jax.ShapeDtypeStruct((2 * EP, H), jnp.float32),
        mesh=_sc_mesh(),
        scratch_types=[
            pltpu.VMEM((4, K2), jnp.int32),
            pltpu.VMEM((4, K2), jnp.int32),
            pltpu.VMEM((2, K2, H), jnp.float32),
            pltpu.VMEM((2, K2, H), jnp.float32),
            pltpu.SemaphoreType.DMA,
            pltpu.SemaphoreType.DMA,
            pltpu.SemaphoreType.DMA,
            pltpu.SemaphoreType.DMA,
            pltpu.SemaphoreType.DMA,
            pltpu.SemaphoreType.DMA,
            pltpu.SemaphoreType.DMA,
            pltpu.SemaphoreType.DMA,
        ],
    )
    def gat(a_hbm, b_hbm, s_hbm, d_hbm, out_hbm,
            si_v, di_v, ra_v, rb_v,
            xi0, xi1, xi2, xi3, sa0, sa1, sb0, sb1):
        cid = lax.axis_index("c")
        sid = lax.axis_index("s")
        wid = sid * NC + cid
        xis = (xi0, xi1, xi2, xi3)
        sas = (sa0, sa1)
        sbs = (sb0, sb1)

        def issue_idx(c, s):
            base = wid * EW2 + c * K2
            pltpu.async_copy(s_hbm.at[pl.ds(base, K2)], si_v.at[s], xis[s])
            pltpu.async_copy(d_hbm.at[pl.ds(base, K2)], di_v.at[s], xis[s])

        def wait_idx(s):
            pltpu.make_async_copy(s_hbm.at[pl.ds(0, K2)], si_v.at[s],
                                  xis[s]).wait()
            pltpu.make_async_copy(d_hbm.at[pl.ds(0, K2)], di_v.at[s],
                                  xis[s]).wait()

        def issue_data(i, d):
            pltpu.async_copy(a_hbm.at[si_v.at[i]], ra_v.at[d], sas[d])
            pltpu.async_copy(b_hbm.at[di_v.at[i]], rb_v.at[d], sbs[d])

        def wait_data(i, d):
            pltpu.make_async_copy(a_hbm.at[si_v.at[i]], ra_v.at[d],
                                  sas[d]).wait()
            pltpu.make_async_copy(b_hbm.at[di_v.at[i]], rb_v.at[d],
                                  sbs[d]).wait()

        def process(c, d):
            base = wid * EW2 + c * K2

            def rbody(r, _):
                for g in range(GROUPS):
                    sl = pl.ds(g * 16, 16)
                    ra_v[d, r, sl] = ra_v[d, r, sl] + rb_v[d, r, sl]
                return 0
            lax.fori_loop(0, K2, rbody, 0)
            pltpu.sync_copy(ra_v.at[d], out_hbm.at[pl.ds(base, K2)])

        NQ2 = NCH2 // 4
        for s in range(4):
            pltpu.sync_copy(s_hbm.at[pl.ds(wid * EW2 + s * K2, K2)],
                            si_v.at[s])
            pltpu.sync_copy(d_hbm.at[pl.ds(wid * EW2 + s * K2, K2)],
                            di_v.at[s])
        issue_data(0, 0)

        def quad(u, _):
            c0 = u * 4
            more = u < NQ2 - 1
            issue_data(1, 1)
            wait_data(0, 0)
            process(c0, 0)

            @pl.when(more)
            def _():
                issue_idx(c0 + 4, 0)
            issue_data(2, 0)
            wait_data(1, 1)
            process(c0 + 1, 1)

            @pl.when(more)
            def _():
                issue_idx(c0 + 5, 1)
            issue_data(3, 1)
            wait_data(2, 0)
            process(c0 + 2, 0)

            @pl.when(more)
            def _():
                issue_idx(c0 + 6, 2)
                wait_idx(0)
                issue_data(0, 0)
            wait_data(3, 1)
            process(c0 + 3, 1)

            @pl.when(more)
            def _():
                issue_idx(c0 + 7, 3)
                wait_idx(1)
                wait_idx(2)
                wait_idx(3)
            return 0
        lax.fori_loop(0, NQ2, quad, 0)
    return gat


def _mm_body(x_ref, w_ref, b_ref, o_ref):
    y = (jnp.dot(x_ref[...], w_ref[...],
                 preferred_element_type=jnp.float32) + b_ref[...])
    o_ref[...] = y.astype(o_ref.dtype)


def _mm(x, w, b, bm, out_dtype=jnp.float32):
    m, k = x.shape
    n = w.shape[1]
    return pl.pallas_call(
        _mm_body,
        grid=(m // bm,),
        in_specs=[pl.BlockSpec((bm, k), lambda i: (i, 0)),
                  pl.BlockSpec((k, n), lambda i: (0, 0)),
                  pl.BlockSpec((1, n), lambda i: (0, 0))],
        out_specs=pl.BlockSpec((bm, n), lambda i: (i, 0)),
        out_shape=jax.ShapeDtypeStruct((m, n), out_dtype),
    )(x, w, b)


def _embed_body(xe_ref, we_ref, be_ref, xn_ref, wn_ref, bn_ref,
                ea_ref, h_ref):
    ea_ref[...] = (jnp.dot(xe_ref[...], we_ref[...],
                           preferred_element_type=jnp.float32) + be_ref[...])
    h_ref[...] = (jnp.dot(xn_ref[...], wn_ref[...],
                          preferred_element_type=jnp.float32) + bn_ref[...])


def _embed(edge_attr, we, be, x, wn, bn):
    GB = 25
    be_, bn_ = E // GB, N // GB
    return pl.pallas_call(
        _embed_body,
        grid=(GB,),
        in_specs=[pl.BlockSpec((be_, DE), lambda i: (i, 0)),
                  pl.BlockSpec((DE, H), lambda i: (0, 0)),
                  pl.BlockSpec((1, H), lambda i: (0, 0)),
                  pl.BlockSpec((bn_, H), lambda i: (i, 0)),
                  pl.BlockSpec((H, H), lambda i: (0, 0)),
                  pl.BlockSpec((1, H), lambda i: (0, 0))],
        out_specs=(pl.BlockSpec((be_, H), lambda i: (i, 0)),
                   pl.BlockSpec((bn_, H), lambda i: (i, 0))),
        out_shape=(jax.ShapeDtypeStruct((E, H), jnp.float32),
                   jax.ShapeDtypeStruct((N, H), jnp.float32)),
    )(edge_attr, we, be, x, wn, bn)


def _layer_body(h_ref, p_ref, w1_ref, b1_ref, w2_ref, b2_ref, g_ref, bb_ref,
                o_ref):
    h = h_ref[...]
    z0 = h + p_ref[0:N, :] + p_ref[N:2 * N, :]
    z1 = jnp.maximum(jnp.dot(z0, w1_ref[...],
                             preferred_element_type=jnp.float32) + b1_ref[...],
                     0.0)
    z2 = jnp.dot(z1, w2_ref[...],
                 preferred_element_type=jnp.float32) + b2_ref[...]
    mu = jnp.mean(z2, axis=0, keepdims=True)
    var = jnp.mean((z2 - mu) ** 2, axis=0, keepdims=True)
    zn = (z2 - mu) / jnp.sqrt(var + 1e-5) * g_ref[...] + bb_ref[...]
    o_ref[...] = (h + jnp.maximum(zn, 0.0)) * 0.5


def _layer(h, parts, w1, b1, w2, b2, g, bb):
    return pl.pallas_call(
        _layer_body,
        out_shape=jax.ShapeDtypeStruct((N, H), jnp.float32),
    )(h, parts, w1, b1[None, :], w2, b2[None, :], g[None, :], bb[None, :])


def _layer_ab_body(h_ref, p_ref, w1_ref, b1_ref, w2_ref, b2_ref, g_ref,
                   bb_ref, wa_ref, wb_ref, o_ref, a_ref, b_ref):
    h = h_ref[...]
    z0 = h + p_ref[0:N, :] + p_ref[N:2 * N, :]
    z1 = jnp.maximum(jnp.dot(z0, w1_ref[...],
                             preferred_element_type=jnp.float32) + b1_ref[...],
                     0.0)
    z2 = jnp.dot(z1, w2_ref[...],
                 preferred_element_type=jnp.float32) + b2_ref[...]
    mu = jnp.mean(z2, axis=0, keepdims=True)
    var = jnp.mean((z2 - mu) ** 2, axis=0, keepdims=True)
    zn = (z2 - mu) / jnp.sqrt(var + 1e-5) * g_ref[...] + bb_ref[...]
    hn = (h + jnp.maximum(zn, 0.0)) * 0.5
    o_ref[...] = hn
    hr = jnp.maximum(hn, 0.0)
    a_ref[...] = jnp.dot(hr, wa_ref[...], preferred_element_type=jnp.float32)
    b_ref[...] = jnp.dot(hr, wb_ref[...], preferred_element_type=jnp.float32)


def _layer_ab(h, parts, w1, b1, w2, b2, g, bb, wa, wb):
    return pl.pallas_call(
        _layer_ab_body,
        out_shape=(jax.ShapeDtypeStruct((N, H), jnp.float32),
                   jax.ShapeDtypeStruct((N, H), jnp.float32),
                   jax.ShapeDtypeStruct((N, H), jnp.float32)),
    )(h, parts, w1, b1[None, :], w2, b2[None, :], g[None, :], bb[None, :],
      wa, wb)


def _head_body(g_ref, at_ref, we_ref, be_ref, w1c_ref, rb1_ref,
               w2_ref, b2_ref, w3_ref, b3_ref, o_ref):
    wcp = jnp.dot(we_ref[...], w1c_ref[...],
                  preferred_element_type=jnp.float32)
    bcp = jnp.dot(be_ref[...], w1c_ref[...],
                  preferred_element_type=jnp.float32) + rb1_ref[...]
    c = jnp.dot(at_ref[...], wcp, preferred_element_type=jnp.float32) + bcp
    z = jnp.maximum(g_ref[...] + c, 0.0)
    z = jnp.maximum(jnp.dot(z, w2_ref[...],
                            preferred_element_type=jnp.float32) + b2_ref[...],
                    0.0)
    s = jnp.dot(z, w3_ref[...], preferred_element_type=jnp.float32) + b3_ref[...]
    o_ref[...] = 1.0 / (1.0 + jnp.exp(-s))


def _head(g, attr2, we, be, w1c, rb1, w2p, b2p, w3p, b3p, bm):
    m = g.shape[0]
    return pl.pallas_call(
        _head_body,
        grid=(m // bm,),
        in_specs=[pl.BlockSpec((bm, H), lambda i: (i, 0)),
                  pl.BlockSpec((bm, DE), lambda i: (i, 0)),
                  pl.BlockSpec((DE, H), lambda i: (0, 0)),
                  pl.BlockSpec((1, H), lambda i: (0, 0)),
                  pl.BlockSpec((H, H), lambda i: (0, 0)),
                  pl.BlockSpec((1, H), lambda i: (0, 0)),
                  pl.BlockSpec((H, 32), lambda i: (0, 0)),
                  pl.BlockSpec((1, 32), lambda i: (0, 0)),
                  pl.BlockSpec((32, 8), lambda i: (0, 0)),
                  pl.BlockSpec((1, 8), lambda i: (0, 0))],
        out_specs=pl.BlockSpec((bm, 8), lambda i: (i, 0)),
        out_shape=jax.ShapeDtypeStruct((m, 8), jnp.float32),
    )(g, attr2, we, be, w1c, rb1, w2p, b2p, w3p, b3p)


@jax.jit
def kernel(x, edge_index, edge_attr, pos_edge_index, pos_edge_attr,
           neg_edge_index, neg_edge_attr, params):
    p = params
    ea, h = _embed(edge_attr, p['edge_W'], p['edge_b'][None, :],
                   x, p['node_W'], p['node_b'][None, :])
    src = edge_index[0]
    dst = edge_index[1]

    w1a = p['r_W1'][0:H, :]
    w1b = p['r_W1'][H:2 * H, :]
    w1c = p['r_W1'][2 * H:3 * H, :]
    attr2 = jnp.concatenate([pos_edge_attr, neg_edge_attr], axis=0)

    msg = _sc_message_kernel()
    parts = msg(h, ea, src, dst)
    h = _layer(h, parts, p['conv0_W1'], p['conv0_b1'],
               p['conv0_W2'], p['conv0_b2'], p['bn0_g'], p['bn0_b'])
    parts = msg(h, ea, src, dst)
    h, a, b = _layer_ab(h, parts, p['conv1_W1'], p['conv1_b1'],
                        p['conv1_W2'], p['conv1_b2'], p['bn1_g'], p['bn1_b'],
                        w1a, w1b)

    ei2 = jnp.concatenate([pos_edge_index, neg_edge_index], axis=1)
    g = _sc_gather2_kernel()(a, b, ei2[0], ei2[1])

    w2p = jnp.pad(p['r_W2'], ((0, 0), (0, 7)))
    b2p = jnp.pad(p['r_b2'], (0, 7))[None, :]
    w3p = jnp.pad(p['r_W3'], ((0, 7), (0, 7)))
    b3p = jnp.pad(p['r_b3'], (0, 7))[None, :]
    out = _head(g, attr2, p['edge_W'], p['edge_b'][None, :], w1c,
                p['r_b1'][None, :], w2p, b2p, w3p, b3p, 8192)
    return out[:EP, 0:1], out[EP:, 0:1], h


# speedup vs baseline: 5.0198x; 1.0002x over previous
import functools

import jax
import jax.numpy as jnp
from jax import lax
from jax.experimental import pallas as pl
from jax.experimental.pallas import tpu as pltpu
from jax.experimental.pallas import tpu_sc as plsc

N = 10000
E = 320000
EP = 65536
H = 128
DE = 16

NC, NS = 2, 16
NW = NC * NS
EW = E // NW
KE = 40
NCH = EW // KE
GROUPS = H // 16
ROWCH = 80
NZCH = N // ROWCH
ZITER = (NZCH + NS - 1) // NS

K2 = 128
EW2 = 2 * EP // NW
NCH2 = EW2 // K2


@functools.lru_cache(maxsize=None)
def _sc_mesh():
    return plsc.VectorSubcoreMesh(core_axis_name="c", subcore_axis_name="s",
                                  num_cores=NC, num_subcores=NS)


@functools.lru_cache(maxsize=None)
def _sc_message_kernel():
    @functools.partial(
        pl.kernel,
        out_type=jax.ShapeDtypeStruct((NC * N, H), jnp.float32),
        mesh=_sc_mesh(),
        scratch_types=[
            pltpu.VMEM((8, KE), jnp.int32),
            pltpu.VMEM((8, KE), jnp.int32),
            pltpu.VMEM((4, KE, H), jnp.float32),
            pltpu.VMEM((4, KE, H), jnp.float32),
            pltpu.VMEM_SHARED((N, H), jnp.float32),
        ] + [pltpu.SemaphoreType.DMA] * 20,
    )
    def msg(h_hbm, ea_hbm, src_hbm, dst_hbm, out_hbm,
            src_v, dst_v, rows_v, ea_v, acc_sh, *sems):
        cid = lax.axis_index("c")
        sid = lax.axis_index("s")
        wid = sid * NC + cid
        sis = sems[0:8]
        sgs = sems[8:12]
        ses = sems[12:16]
        scs = sems[16:20]

        def zrow(r, _):
            for g in range(GROUPS):
                rows_v[0, r, pl.ds(g * 16, 16)] = jnp.zeros((16,), jnp.float32)
            return 0
        lax.fori_loop(0, KE, zrow, 0)

        def zchunk(t, _):
            c = sid + t * NS

            @pl.when(c < NZCH)
            def _():
                for rep in range(ROWCH // KE):
                    pltpu.sync_copy(
                        rows_v.at[0],
                        acc_sh.at[pl.ds(c * ROWCH + rep * KE, KE)])
            return 0
        lax.fori_loop(0, ZITER, zchunk, 0)
        plsc.subcore_barrier()

        def issue_idx(c, s):
            base = wid * EW + c * KE
            pltpu.async_copy(src_hbm.at[pl.ds(base, KE)], src_v.at[s], sis[s])
            pltpu.async_copy(dst_hbm.at[pl.ds(base, KE)], dst_v.at[s], sis[s])

        def wait_idx(s):
            pltpu.make_async_copy(src_hbm.at[pl.ds(0, KE)], src_v.at[s],
                                  sis[s]).wait()
            pltpu.make_async_copy(dst_hbm.at[pl.ds(0, KE)], dst_v.at[s],
                                  sis[s]).wait()

        def issue_data(c, si, d):
            base = wid * EW + c * KE
            pltpu.async_copy(h_hbm.at[src_v.at[si]], rows_v.at[d], sgs[d])
            pltpu.async_copy(ea_hbm.at[pl.ds(base, KE)], ea_v.at[d], ses[d])

        def wait_data(si, d):
            pltpu.make_async_copy(h_hbm.at[src_v.at[si]], rows_v.at[d],
                                  sgs[d]).wait()
            pltpu.make_async_copy(ea_hbm.at[pl.ds(0, KE)], ea_v.at[d],
                                  ses[d]).wait()

        def process(si, d):
            def rbody(r, _):
                for g in range(GROUPS):
                    sl = pl.ds(g * 16, 16)
                    rows_v[d, r, sl] = jnp.maximum(
                        rows_v[d, r, sl] + ea_v[d, r, sl], 0.0)
                return 0
            lax.fori_loop(0, KE, rbody, 0)
            pltpu.async_copy(rows_v.at[d], acc_sh.at[dst_v.at[si]], scs[d],
                             add=True)

        def wait_scatter(si, d):
            pltpu.make_async_copy(rows_v.at[d], acc_sh.at[dst_v.at[si]],
                                  scs[d]).wait()

        def steady(c, si, d, first=False):
            wait_data(si, d)
            process(si, d)
            if not first:
                wait_scatter((si + 6) % 8, (d + 2) % 4)
            issue_idx(c + 6, (si + 6) % 8)
            wait_idx((si + 2) % 8)
            issue_data(c + 2, (si + 2) % 8, (d + 2) % 4)

        pltpu.sync_copy(src_hbm.at[pl.ds(wid * EW, KE)], src_v.at[0])
        pltpu.sync_copy(dst_hbm.at[pl.ds(wid * EW, KE)], dst_v.at[0])
        pltpu.sync_copy(src_hbm.at[pl.ds(wid * EW + KE, KE)], src_v.at[1])
        pltpu.sync_copy(dst_hbm.at[pl.ds(wid * EW + KE, KE)], dst_v.at[1])
        for s in range(2, 6):
            issue_idx(s, s)
        issue_data(0, 0, 0)
        issue_data(1, 1, 1)

        for c in range(2):
            steady(c, c, c, first=True)
        for c in range(2, 8):
            steady(c, c, c % 4)

        def octad(u, _):
            c0 = u * 8
            for j in range(8):
                steady(c0 + j, j, j % 4)
            return 0
        NSTEADY = (NCH - 6 - 8) // 8
        lax.fori_loop(1, 1 + NSTEADY, octad, 0)

        TAILC = 8 + NSTEADY * 8
        for c in range(TAILC, NCH):
            si = c % 8
            d = c % 4
            wait_data(si, d)
            process(si, d)
            wait_scatter((si + 6) % 8, (d + 2) % 4)
            if c + 6 < NCH:
                issue_idx(c + 6, (si + 6) % 8)
            if c + 2 < NCH:
                wait_idx((si + 2) % 8)
                issue_data(c + 2, (si + 2) % 8, (d + 2) % 4)
        wait_scatter((NCH - 2) % 8, (NCH - 2) % 4)
        wait_scatter((NCH - 1) % 8, (NCH - 1) % 4)
        plsc.subcore_barrier()

        def wchunk(t, _):
            c = sid + t * NS

            @pl.when(c < NZCH)
            def _():
                off = cid * N + c * ROWCH
                for rep in range(ROWCH // KE):
                    pltpu.sync_copy(
                        acc_sh.at[pl.ds(c * ROWCH + rep * KE, KE)],
                        out_hbm.at[pl.ds(off + rep * KE, KE)])
            return 0
        lax.fori_loop(0, ZITER, wchunk, 0)
    return msg


@functools.lru_cache(maxsize=None)
def _sc_gather2_kernel():
    @functools.partial(
        pl.kernel,
        out_type=jax.ShapeDtypeStruct((2 * EP, H), jnp.float32),
        mesh=_sc_mesh(),
        scratch_types=[
            pltpu.VMEM((4, K2), jnp.int32),
            pltpu.VMEM((4, K2), jnp.int32),
            pltpu.VMEM((2, K2, H), jnp.float32),
            pltpu.VMEM((2, K2, H), jnp.float32),
            pltpu.SemaphoreType.DMA,
            pltpu.SemaphoreType.DMA,
            pltpu.SemaphoreType.DMA,
            pltpu.SemaphoreType.DMA,
            pltpu.SemaphoreType.DMA,
            pltpu.SemaphoreType.DMA,
            pltpu.SemaphoreType.DMA,
            pltpu.SemaphoreType.DMA,
        ],
    )
    def gat(a_hbm, b_hbm, s_hbm, d_hbm, out_hbm,
            si_v, di_v, ra_v, rb_v,
            xi0, xi1, xi2, xi3, sa0, sa1, sb0, sb1):
        cid = lax.axis_index("c")
        sid = lax.axis_index("s")
        wid = sid * NC + cid
        xis = (xi0, xi1, xi2, xi3)
        sas = (sa0, sa1)
        sbs = (sb0, sb1)

        def issue_idx(c, s):
            base = wid * EW2 + c * K2
            pltpu.async_copy(s_hbm.at[pl.ds(base, K2)], si_v.at[s], xis[s])
            pltpu.async_copy(d_hbm.at[pl.ds(base, K2)], di_v.at[s], xis[s])

        def wait_idx(s):
            pltpu.make_async_copy(s_hbm.at[pl.ds(0, K2)], si_v.at[s],
                                  xis[s]).wait()
            pltpu.make_async_copy(d_hbm.at[pl.ds(0, K2)], di_v.at[s],
                                  xis[s]).wait()

        def issue_data(i, d):
            pltpu.async_copy(a_hbm.at[si_v.at[i]], ra_v.at[d], sas[d])
            pltpu.async_copy(b_hbm.at[di_v.at[i]], rb_v.at[d], sbs[d])

        def wait_data(i, d):
            pltpu.make_async_copy(a_hbm.at[si_v.at[i]], ra_v.at[d],
                                  sas[d]).wait()
            pltpu.make_async_copy(b_hbm.at[di_v.at[i]], rb_v.at[d],
                                  sbs[d]).wait()

        def process(c, d):
            base = wid * EW2 + c * K2

            def rbody(r, _):
                for g in range(GROUPS):
                    sl = pl.ds(g * 16, 16)
                    ra_v[d, r, sl] = ra_v[d, r, sl] + rb_v[d, r, sl]
                return 0
            lax.fori_loop(0, K2, rbody, 0)
            pltpu.sync_copy(ra_v.at[d], out_hbm.at[pl.ds(base, K2)])

        NQ2 = NCH2 // 4
        for s in range(4):
            pltpu.sync_copy(s_hbm.at[pl.ds(wid * EW2 + s * K2, K2)],
                            si_v.at[s])
            pltpu.sync_copy(d_hbm.at[pl.ds(wid * EW2 + s * K2, K2)],
                            di_v.at[s])
        issue_data(0, 0)

        def quad(u, _):
            c0 = u * 4
            more = u < NQ2 - 1
            issue_data(1, 1)
            wait_data(0, 0)
            process(c0, 0)

            @pl.when(more)
            def _():
                issue_idx(c0 + 4, 0)
            issue_data(2, 0)
            wait_data(1, 1)
            process(c0 + 1, 1)

            @pl.when(more)
            def _():
                issue_idx(c0 + 5, 1)
            issue_data(3, 1)
            wait_data(2, 0)
            process(c0 + 2, 0)

            @pl.when(more)
            def _():
                issue_idx(c0 + 6, 2)
                wait_idx(0)
                issue_data(0, 0)
            wait_data(3, 1)
            process(c0 + 3, 1)

            @pl.when(more)
            def _():
                issue_idx(c0 + 7, 3)
                wait_idx(1)
                wait_idx(2)
                wait_idx(3)
            return 0
        lax.fori_loop(0, NQ2, quad, 0)
    return gat


def _embed_body(xe_ref, we_ref, be_ref, xn_ref, wn_ref, bn_ref,
                ea_ref, h_ref):
    ea_ref[...] = (jnp.dot(xe_ref[...], we_ref[...],
                           preferred_element_type=jnp.float32) + be_ref[...])
    h_ref[...] = (jnp.dot(xn_ref[...], wn_ref[...],
                          preferred_element_type=jnp.float32) + bn_ref[...])


def _embed(edge_attr, we, be, x, wn, bn):
    GB = 25
    be_, bn_ = E // GB, N // GB
    return pl.pallas_call(
        _embed_body,
        grid=(GB,),
        in_specs=[pl.BlockSpec((be_, DE), lambda i: (i, 0)),
                  pl.BlockSpec((DE, H), lambda i: (0, 0)),
                  pl.BlockSpec((1, H), lambda i: (0, 0)),
                  pl.BlockSpec((bn_, H), lambda i: (i, 0)),
                  pl.BlockSpec((H, H), lambda i: (0, 0)),
                  pl.BlockSpec((1, H), lambda i: (0, 0))],
        out_specs=(pl.BlockSpec((be_, H), lambda i: (i, 0)),
                   pl.BlockSpec((bn_, H), lambda i: (i, 0))),
        out_shape=(jax.ShapeDtypeStruct((E, H), jnp.float32),
                   jax.ShapeDtypeStruct((N, H), jnp.float32)),
    )(edge_attr, we, be, x, wn, bn)


def _layer_body(h_ref, p_ref, w1_ref, b1_ref, w2_ref, b2_ref, g_ref, bb_ref,
                o_ref):
    h = h_ref[...]
    z0 = h + p_ref[0:N, :] + p_ref[N:2 * N, :]
    z1 = jnp.maximum(jnp.dot(z0, w1_ref[...],
                             preferred_element_type=jnp.float32) + b1_ref[...],
                     0.0)
    z2 = jnp.dot(z1, w2_ref[...],
                 preferred_element_type=jnp.float32) + b2_ref[...]
    mu = jnp.mean(z2, axis=0, keepdims=True)
    var = jnp.mean((z2 - mu) ** 2, axis=0, keepdims=True)
    zn = (z2 - mu) / jnp.sqrt(var + 1e-5) * g_ref[...] + bb_ref[...]
    o_ref[...] = (h + jnp.maximum(zn, 0.0)) * 0.5


def _layer(h, parts, w1, b1, w2, b2, g, bb):
    return pl.pallas_call(
        _layer_body,
        out_shape=jax.ShapeDtypeStruct((N, H), jnp.float32),
    )(h, parts, w1, b1[None, :], w2, b2[None, :], g[None, :], bb[None, :])


def _layer_ab_body(h_ref, p_ref, w1_ref, b1_ref, w2_ref, b2_ref, g_ref,
                   bb_ref, wa_ref, wb_ref, o_ref, a_ref, b_ref):
    h = h_ref[...]
    z0 = h + p_ref[0:N, :] + p_ref[N:2 * N, :]
    z1 = jnp.maximum(jnp.dot(z0, w1_ref[...],
                             preferred_element_type=jnp.float32) + b1_ref[...],
                     0.0)
    z2 = jnp.dot(z1, w2_ref[...],
                 preferred_element_type=jnp.float32) + b2_ref[...]
    mu = jnp.mean(z2, axis=0, keepdims=True)
    var = jnp.mean((z2 - mu) ** 2, axis=0, keepdims=True)
    zn = (z2 - mu) / jnp.sqrt(var + 1e-5) * g_ref[...] + bb_ref[...]
    hn = (h + jnp.maximum(zn, 0.0)) * 0.5
    o_ref[...] = hn
    hr = jnp.maximum(hn, 0.0)
    a_ref[...] = jnp.dot(hr, wa_ref[...], preferred_element_type=jnp.float32)
    b_ref[...] = jnp.dot(hr, wb_ref[...], preferred_element_type=jnp.float32)


def _layer_ab(h, parts, w1, b1, w2, b2, g, bb, wa, wb):
    return pl.pallas_call(
        _layer_ab_body,
        out_shape=(jax.ShapeDtypeStruct((N, H), jnp.float32),
                   jax.ShapeDtypeStruct((N, H), jnp.float32),
                   jax.ShapeDtypeStruct((N, H), jnp.float32)),
    )(h, parts, w1, b1[None, :], w2, b2[None, :], g[None, :], bb[None, :],
      wa, wb)


def _head_body(g_ref, at_ref, we_ref, be_ref, w1c_ref, rb1_ref,
               w2_ref, b2_ref, w3_ref, b3_ref, o_ref):
    wcp = jnp.dot(we_ref[...], w1c_ref[...],
                  preferred_element_type=jnp.float32)
    bcp = jnp.dot(be_ref[...], w1c_ref[...],
                  preferred_element_type=jnp.float32) + rb1_ref[...]
    c = jnp.dot(at_ref[...], wcp, preferred_element_type=jnp.float32) + bcp
    z = jnp.maximum(g_ref[...] + c, 0.0)
    z = jnp.maximum(jnp.dot(z, w2_ref[...],
                            preferred_element_type=jnp.float32) + b2_ref[...],
                    0.0)
    s = jnp.dot(z, w3_ref[...], preferred_element_type=jnp.float32) + b3_ref[...]
    o_ref[...] = 1.0 / (1.0 + jnp.exp(-s))


def _head(g, attr2, we, be, w1c, rb1, w2p, b2p, w3p, b3p, bm):
    m = g.shape[0]
    return pl.pallas_call(
        _head_body,
        grid=(m // bm,),
        in_specs=[pl.BlockSpec((bm, H), lambda i: (i, 0)),
                  pl.BlockSpec((bm, DE), lambda i: (i, 0)),
                  pl.BlockSpec((DE, H), lambda i: (0, 0)),
                  pl.BlockSpec((1, H), lambda i: (0, 0)),
                  pl.BlockSpec((H, H), lambda i: (0, 0)),
                  pl.BlockSpec((1, H), lambda i: (0, 0)),
                  pl.BlockSpec((H, 32), lambda i: (0, 0)),
                  pl.BlockSpec((1, 32), lambda i: (0, 0)),
                  pl.BlockSpec((32, 8), lambda i: (0, 0)),
                  pl.BlockSpec((1, 8), lambda i: (0, 0))],
        out_specs=pl.BlockSpec((bm, 8), lambda i: (i, 0)),
        out_shape=jax.ShapeDtypeStruct((m, 8), jnp.float32),
    )(g, attr2, we, be, w1c, rb1, w2p, b2p, w3p, b3p)


@jax.jit
def kernel(x, edge_index, edge_attr, pos_edge_index, pos_edge_attr,
           neg_edge_index, neg_edge_attr, params):
    p = params
    ea, h = _embed(edge_attr, p['edge_W'], p['edge_b'][None, :],
                   x, p['node_W'], p['node_b'][None, :])
    src = edge_index[0]
    dst = edge_index[1]

    w1a = p['r_W1'][0:H, :]
    w1b = p['r_W1'][H:2 * H, :]
    w1c = p['r_W1'][2 * H:3 * H, :]
    attr2 = jnp.concatenate([pos_edge_attr, neg_edge_attr], axis=0)

    msg = _sc_message_kernel()
    parts = msg(h, ea, src, dst)
    h = _layer(h, parts, p['conv0_W1'], p['conv0_b1'],
               p['conv0_W2'], p['conv0_b2'], p['bn0_g'], p['bn0_b'])
    parts = msg(h, ea, src, dst)
    h, a, b = _layer_ab(h, parts, p['conv1_W1'], p['conv1_b1'],
                        p['conv1_W2'], p['conv1_b2'], p['bn1_g'], p['bn1_b'],
                        w1a, w1b)

    ei2 = jnp.concatenate([pos_edge_index, neg_edge_index], axis=1)
    g = _sc_gather2_kernel()(a, b, ei2[0], ei2[1])

    w2p = jnp.pad(p['r_W2'], ((0, 0), (0, 7)))
    b2p = jnp.pad(p['r_b2'], (0, 7))[None, :]
    w3p = jnp.pad(p['r_W3'], ((0, 7), (0, 7)))
    b3p = jnp.pad(p['r_b3'], (0, 7))[None, :]
    out = _head(g, attr2, p['edge_W'], p['edge_b'][None, :], w1c,
                p['r_b1'][None, :], w2p, b2p, w3p, b3p, 8192)
    return out[:EP, 0:1], out[EP:, 0:1], h
